# baseline (speedup 1.0000x reference)
import functools

import jax
import jax.numpy as jnp
from jax import lax
from jax.experimental import pallas as pl
from jax.experimental.pallas import tpu as pltpu
from jax.experimental.pallas import tpu_sc as plsc

NC = 2
NS = 16
NW = NC * NS

CH = 80
NB = 4


def _sc_mesh():
  return plsc.VectorSubcoreMesh(
      core_axis_name="c", subcore_axis_name="s", num_cores=NC,
      num_subcores=NS)


_SC_PARAMS = pltpu.CompilerParams(
    use_tc_tiling_on_sc=False, needs_layout_passes=False)

def _shuffle_add(v, iot):
  dnums = lax.GatherDimensionNumbers(
      offset_dims=(), collapsed_slice_dims=(0,), start_index_map=(0,))
  for sh in (8, 4, 2, 1):
    idx = jnp.bitwise_xor(iot, jnp.int32(sh)).reshape(16, 1)
    v = v + lax.gather(v, idx, dnums, slice_sizes=(1,),
                       mode=lax.GatherScatterMode.PROMISE_IN_BOUNDS)
  return v


def _zero_2d(ref, rows, width):
  zv = jnp.zeros((16,), jnp.float32)

  @pl.loop(0, rows)
  def _(r):
    for k in range(width // 16):
      ref[r, pl.ds(k * 16, 16)] = zv


def _make_segsum(n_nodes, n_edges, width, nb=3):
  ept = n_edges // NW
  nchunk = ept // CH
  n_outer = (nchunk + nb - 1) // nb
  rows_pt = n_nodes // NS
  nfull, tail = divmod(rows_pt, CH)

  scratch = (
      [pltpu.VMEM((nb, CH), jnp.int32)] * 2
      + [pltpu.VMEM((CH, width), jnp.float32) for _ in range(nb)]
      + [pltpu.VMEM_SHARED((n_nodes, width), jnp.float32)]
      + [pltpu.SemaphoreType.DMA] * (3 * nb)
  )

  @functools.partial(
      pl.kernel,
      out_type=jax.ShapeDtypeStruct((NC * n_nodes, width), jnp.float32),
      mesh=_sc_mesh(),
      scratch_types=scratch,
      compiler_params=_SC_PARAMS,
  )
  def segsum(y_hbm, src2d, dst2d, out_hbm, sring, dring, *rest):
    bufs = rest[:nb]
    agg_sh = rest[nb]
    gsem = rest[nb + 1:2 * nb + 1]
    isems = rest[2 * nb + 1:3 * nb + 1]
    isemd = rest[3 * nb + 1:]
    c = lax.axis_index("c")
    s = lax.axis_index("s")
    wid = c * NS + s
    cbase = wid * nchunk

    _zero_2d(bufs[0], CH, width)
    for r in range(nfull):
      pltpu.sync_copy(bufs[0], agg_sh.at[pl.ds(s * rows_pt + r * CH, CH)])
    if tail:
      pltpu.sync_copy(bufs[0].at[pl.ds(0, tail)],
                      agg_sh.at[pl.ds(s * rows_pt + nfull * CH, tail)])
    plsc.subcore_barrier()

    for b in range(nb):
      pltpu.sync_copy(src2d.at[cbase + b], sring.at[b])
      pltpu.sync_copy(dst2d.at[cbase + b], dring.at[b])
      pltpu.async_copy(y_hbm.at[sring.at[b]], bufs[b], gsem[b])

    @pl.loop(0, n_outer)
    def _(o):
      for b in range(nb):
        i = o * nb + b
        b1 = (b + 1) % nb

        @pl.when(i < nchunk)
        def _(i=i, b=b, b1=b1):
          pltpu.make_async_copy(y_hbm.at[sring.at[b]], bufs[b],
                                gsem[b]).wait()
          j = i + 1

          @pl.when(jnp.logical_and(j >= nb, j < nchunk))
          def _():
            pltpu.make_async_copy(src2d.at[cbase + j], sring.at[b1],
                                  isems[b1]).wait()
            pltpu.make_async_copy(dst2d.at[cbase + j], dring.at[b1],
                                  isemd[b1]).wait()
            pltpu.async_copy(y_hbm.at[sring.at[b1]], bufs[b1], gsem[b1])

          pltpu.sync_copy(bufs[b], agg_sh.at[dring.at[b]], add=True)

          @pl.when(i + nb < nchunk)
          def _():
            pltpu.async_copy(src2d.at[cbase + i + nb], sring.at[b], isems[b])
            pltpu.async_copy(dst2d.at[cbase + i + nb], dring.at[b], isemd[b])

    plsc.subcore_barrier()

    for r in range(nfull):
      row0 = s * rows_pt + r * CH
      pltpu.sync_copy(agg_sh.at[pl.ds(row0, CH)], bufs[0])
      pltpu.sync_copy(bufs[0], out_hbm.at[pl.ds(c * n_nodes + row0, CH)])
    if tail:
      row0 = s * rows_pt + nfull * CH
      pltpu.sync_copy(agg_sh.at[pl.ds(row0, tail)],
                      bufs[0].at[pl.ds(0, tail)])
      pltpu.sync_copy(bufs[0].at[pl.ds(0, tail)],
                      out_hbm.at[pl.ds(c * n_nodes + row0, tail)])

  return segsum


def _make_score(n_nodes, n_edges, width, nb=3):
  ept = n_edges // NW
  nchunk = ept // CH
  n_outer = (nchunk + nb - 1) // nb

  scratch = (
      [pltpu.VMEM((nb, CH), jnp.int32)] * 2
      + [pltpu.VMEM((CH, width), jnp.float32) for _ in range(2 * nb)]
      + [pltpu.VMEM((nchunk, CH), jnp.float32)]
      + [pltpu.SemaphoreType.DMA] * (4 * nb)
  )

  @functools.partial(
      pl.kernel,
      out_type=jax.ShapeDtypeStruct((n_edges // CH, CH), jnp.float32),
      mesh=_sc_mesh(),
      scratch_types=scratch,
      compiler_params=_SC_PARAMS,
  )
  def score(h_hbm, src2d, dst2d, out_hbm, sring, dring, *rest):
    sbufs = rest[:nb]
    dbufs = rest[nb:2 * nb]
    out_all = rest[2 * nb]
    gsems = rest[2 * nb + 1:3 * nb + 1]
    gsemd = rest[3 * nb + 1:4 * nb + 1]
    isems = rest[4 * nb + 1:5 * nb + 1]
    isemd = rest[5 * nb + 1:]
    c = lax.axis_index("c")
    s = lax.axis_index("s")
    wid = c * NS + s
    cbase = wid * nchunk
    iot = lax.iota(jnp.int32, 16)

    for b in range(nb):
      pltpu.sync_copy(src2d.at[cbase + b], sring.at[b])
      pltpu.sync_copy(dst2d.at[cbase + b], dring.at[b])
      pltpu.async_copy(h_hbm.at[sring.at[b]], sbufs[b], gsems[b])
      pltpu.async_copy(h_hbm.at[dring.at[b]], dbufs[b], gsemd[b])

    @pl.loop(0, n_outer)
    def _(o):
      for b in range(nb):
        i = o * nb + b
        b1 = (b + 1) % nb

        @pl.when(i < nchunk)
        def _(i=i, b=b, b1=b1):
          pltpu.make_async_copy(h_hbm.at[sring.at[b]], sbufs[b],
                                gsems[b]).wait()
          pltpu.make_async_copy(h_hbm.at[dring.at[b]], dbufs[b],
                                gsemd[b]).wait()
          j = i + 1

          @pl.when(jnp.logical_and(j >= nb, j < nchunk))
          def _():
            pltpu.make_async_copy(src2d.at[cbase + j], sring.at[b1],
                                  isems[b1]).wait()
            pltpu.make_async_copy(dst2d.at[cbase + j], dring.at[b1],
                                  isemd[b1]).wait()
            pltpu.async_copy(h_hbm.at[sring.at[b1]], sbufs[b1], gsems[b1])
            pltpu.async_copy(h_hbm.at[dring.at[b1]], dbufs[b1], gsemd[b1])

          @plsc.parallel_loop(0, CH // 16)
          def _(g):
            score_vec = jnp.zeros((16,), jnp.float32)
            for el in range(16):
              e = g * 16 + el
              acc = []
              for k in range(width // 16):
                sv = sbufs[b][e, pl.ds(k * 16, 16)]
                dv = dbufs[b][e, pl.ds(k * 16, 16)]
                acc.append(sv * dv)
              while len(acc) > 1:
                acc = [a + c for a, c in zip(acc[::2], acc[1::2])]
              v = _shuffle_add(acc[0], iot)
              score_vec = jnp.where(iot == el, v, score_vec)
            out_all[i, pl.ds(g * 16, 16)] = score_vec

          @pl.when(i + nb < nchunk)
          def _():
            pltpu.async_copy(src2d.at[cbase + i + nb], sring.at[b], isems[b])
            pltpu.async_copy(dst2d.at[cbase + i + nb], dring.at[b], isemd[b])

    pltpu.sync_copy(out_all, out_hbm.at[pl.ds(cbase, nchunk)])

  return score


def _tc1(x, wn1):
  n, din = x.shape
  blk = 1000

  def body(x_ref, w_ref, o_ref):
    y = jnp.dot(x_ref[...], w_ref[...], preferred_element_type=jnp.float32)
    pad_iota = lax.broadcasted_iota(jnp.int32, (blk, 16), 1)
    extra = jnp.where(pad_iota == 0, 1.0, 0.0).astype(jnp.float32)
    o_ref[...] = jnp.concatenate([y, extra], axis=1)

  return pl.pallas_call(
      body,
      grid=(n // blk,),
      in_specs=[
          pl.BlockSpec((blk, din), lambda i: (i, 0)),
          pl.BlockSpec((din, 128), lambda i: (0, 0)),
      ],
      out_specs=pl.BlockSpec((blk, 144), lambda i: (i, 0)),
      out_shape=jax.ShapeDtypeStruct((n, 144), jnp.float32),
  )(x, wn1)


def _tc2(x, ws1, b1, wn2, agg1):
  n, din = x.shape
  blk = 1000

  def body(x_ref, ws_ref, b_ref, wn2_ref, agg_ref, h_ref, y2_ref, inv_ref):
    agg = agg_ref[0] + agg_ref[1]
    deg = agg[:, 128:129]
    inv = 1.0 / jnp.maximum(deg, 1.0)
    mean = agg[:, :128] * inv
    h = jnp.dot(x_ref[...], ws_ref[...], preferred_element_type=jnp.float32)
    h = jnp.maximum(h + mean + b_ref[...], 0.0)
    h_ref[...] = h
    y2_ref[...] = jnp.dot(h, wn2_ref[...], preferred_element_type=jnp.float32)
    inv_ref[...] = jnp.broadcast_to(inv, (blk, 128))

  return pl.pallas_call(
      body,
      grid=(n // blk,),
      in_specs=[
          pl.BlockSpec((blk, din), lambda i: (i, 0)),
          pl.BlockSpec((din, 128), lambda i: (0, 0)),
          pl.BlockSpec((1, 128), lambda i: (0, 0)),
          pl.BlockSpec((128, 128), lambda i: (0, 0)),
          pl.BlockSpec((2, blk, 144), lambda i: (0, i, 0)),
      ],
      out_specs=[
          pl.BlockSpec((blk, 128), lambda i: (i, 0)),
          pl.BlockSpec((blk, 128), lambda i: (i, 0)),
          pl.BlockSpec((blk, 128), lambda i: (i, 0)),
      ],
      out_shape=[
          jax.ShapeDtypeStruct((n, 128), jnp.float32),
          jax.ShapeDtypeStruct((n, 128), jnp.float32),
          jax.ShapeDtypeStruct((n, 128), jnp.float32),
      ],
  )(x, ws1, b1, wn2, agg1)


def _tc3(h1, ws2, b2, agg2, inv):
  n, din = h1.shape
  blk = 1000

  def body(h1_ref, ws_ref, b_ref, agg_ref, inv_ref, o_ref):
    agg = agg_ref[0] + agg_ref[1]
    h = jnp.dot(h1_ref[...], ws_ref[...], preferred_element_type=jnp.float32)
    o_ref[...] = h + agg * inv_ref[...] + b_ref[...]

  return pl.pallas_call(
      body,
      grid=(n // blk,),
      in_specs=[
          pl.BlockSpec((blk, din), lambda i: (i, 0)),
          pl.BlockSpec((din, 128), lambda i: (0, 0)),
          pl.BlockSpec((1, 128), lambda i: (0, 0)),
          pl.BlockSpec((2, blk, 128), lambda i: (0, i, 0)),
          pl.BlockSpec((blk, 128), lambda i: (i, 0)),
      ],
      out_specs=pl.BlockSpec((blk, 128), lambda i: (i, 0)),
      out_shape=jax.ShapeDtypeStruct((n, 128), jnp.float32),
  )(h1, ws2, b2, agg2, inv)


def kernel(features, edge_index, W_self1, W_neigh1, b1, W_self2, W_neigh2,
           b2):
  n, _ = features.shape
  n_edges = edge_index.shape[1]
  src = edge_index[0].astype(jnp.int32).reshape(n_edges // CH, CH)
  dst = edge_index[1].astype(jnp.int32).reshape(n_edges // CH, CH)

  segsum_ext = _make_segsum(n, n_edges, 144)
  segsum = _make_segsum(n, n_edges, 128)
  score_k = _make_score(n, n_edges, 128)

  y1e = _tc1(features, W_neigh1)
  agg1 = segsum_ext(y1e, src, dst).reshape(2, n, 144)
  h1, y2, inv = _tc2(features, W_self1, b1.reshape(1, 128), W_neigh2, agg1)
  agg2 = segsum(y2, src, dst).reshape(2, n, 128)
  h2 = _tc3(h1, W_self2, b2.reshape(1, 128), agg2, inv)
  return score_k(h2, src, dst).reshape(n_edges)


# speedup vs baseline: 1.0208x; 1.0208x over previous
import functools

import jax
import jax.numpy as jnp
from jax import lax
from jax.experimental import pallas as pl
from jax.experimental.pallas import tpu as pltpu
from jax.experimental.pallas import tpu_sc as plsc

NC = 2
NS = 16
NW = NC * NS

CH = 80
NB = 4


def _sc_mesh():
  return plsc.VectorSubcoreMesh(
      core_axis_name="c", subcore_axis_name="s", num_cores=NC,
      num_subcores=NS)


_SC_PARAMS = pltpu.CompilerParams(
    use_tc_tiling_on_sc=False, needs_layout_passes=False)

def _shuffle_add(v, iot):
  dnums = lax.GatherDimensionNumbers(
      offset_dims=(), collapsed_slice_dims=(0,), start_index_map=(0,))
  for sh in (8, 4, 2, 1):
    idx = jnp.bitwise_xor(iot, jnp.int32(sh)).reshape(16, 1)
    v = v + lax.gather(v, idx, dnums, slice_sizes=(1,),
                       mode=lax.GatherScatterMode.PROMISE_IN_BOUNDS)
  return v


def _zero_2d(ref, rows, width):
  zv = jnp.zeros((16,), jnp.float32)

  @pl.loop(0, rows)
  def _(r):
    for k in range(width // 16):
      ref[r, pl.ds(k * 16, 16)] = zv


def _make_segsum(n_nodes, n_edges, width, nb=3):
  ept = n_edges // NW
  nchunk = ept // CH
  n_outer = (nchunk + nb - 1) // nb
  rows_pt = n_nodes // NS
  nfull, tail = divmod(rows_pt, CH)

  scratch = (
      [pltpu.VMEM((nb, CH), jnp.int32)] * 2
      + [pltpu.VMEM((CH, width), jnp.float32) for _ in range(nb)]
      + [pltpu.VMEM_SHARED((n_nodes, width), jnp.float32)]
      + [pltpu.SemaphoreType.DMA] * (3 * nb)
  )

  @functools.partial(
      pl.kernel,
      out_type=jax.ShapeDtypeStruct((NC * n_nodes, width), jnp.float32),
      mesh=_sc_mesh(),
      scratch_types=scratch,
      compiler_params=_SC_PARAMS,
  )
  def segsum(y_hbm, src2d, dst2d, out_hbm, sring, dring, *rest):
    bufs = rest[:nb]
    agg_sh = rest[nb]
    gsem = rest[nb + 1:2 * nb + 1]
    isems = rest[2 * nb + 1:3 * nb + 1]
    isemd = rest[3 * nb + 1:]
    c = lax.axis_index("c")
    s = lax.axis_index("s")
    wid = c * NS + s
    cbase = wid * nchunk

    _zero_2d(bufs[0], CH, width)
    for r in range(nfull):
      pltpu.sync_copy(bufs[0], agg_sh.at[pl.ds(s * rows_pt + r * CH, CH)])
    if tail:
      pltpu.sync_copy(bufs[0].at[pl.ds(0, tail)],
                      agg_sh.at[pl.ds(s * rows_pt + nfull * CH, tail)])
    plsc.subcore_barrier()

    for b in range(nb):
      pltpu.sync_copy(src2d.at[cbase + b], sring.at[b])
      pltpu.sync_copy(dst2d.at[cbase + b], dring.at[b])
      pltpu.async_copy(y_hbm.at[sring.at[b]], bufs[b], gsem[b])

    @pl.loop(0, n_outer)
    def _(o):
      for b in range(nb):
        i = o * nb + b
        b1 = (b + 1) % nb

        @pl.when(i < nchunk)
        def _(i=i, b=b, b1=b1):
          pltpu.make_async_copy(y_hbm.at[sring.at[b]], bufs[b],
                                gsem[b]).wait()
          j = i + 1

          @pl.when(jnp.logical_and(j >= nb, j < nchunk))
          def _():
            pltpu.make_async_copy(src2d.at[cbase + j], sring.at[b1],
                                  isems[b1]).wait()
            pltpu.make_async_copy(dst2d.at[cbase + j], dring.at[b1],
                                  isemd[b1]).wait()
            pltpu.async_copy(y_hbm.at[sring.at[b1]], bufs[b1], gsem[b1])

          pltpu.sync_copy(bufs[b], agg_sh.at[dring.at[b]], add=True)

          @pl.when(i + nb < nchunk)
          def _():
            pltpu.async_copy(src2d.at[cbase + i + nb], sring.at[b], isems[b])
            pltpu.async_copy(dst2d.at[cbase + i + nb], dring.at[b], isemd[b])

    plsc.subcore_barrier()

    for r in range(nfull):
      row0 = s * rows_pt + r * CH
      pltpu.sync_copy(agg_sh.at[pl.ds(row0, CH)], bufs[0])
      pltpu.sync_copy(bufs[0], out_hbm.at[pl.ds(c * n_nodes + row0, CH)])
    if tail:
      row0 = s * rows_pt + nfull * CH
      pltpu.sync_copy(agg_sh.at[pl.ds(row0, tail)],
                      bufs[0].at[pl.ds(0, tail)])
      pltpu.sync_copy(bufs[0].at[pl.ds(0, tail)],
                      out_hbm.at[pl.ds(c * n_nodes + row0, tail)])

  return segsum


def _make_score(n_nodes, n_edges, width, nb=4):
  ept = n_edges // NW
  nchunk = ept // CH
  n_outer = (nchunk + nb - 1) // nb

  scratch = (
      [pltpu.VMEM((nb, CH), jnp.int32)] * 2
      + [pltpu.VMEM((CH, width), jnp.float32) for _ in range(2 * nb)]
      + [pltpu.VMEM((nchunk, CH), jnp.float32)]
      + [pltpu.SemaphoreType.DMA] * (4 * nb)
  )

  @functools.partial(
      pl.kernel,
      out_type=jax.ShapeDtypeStruct((n_edges // CH, CH), jnp.float32),
      mesh=_sc_mesh(),
      scratch_types=scratch,
      compiler_params=_SC_PARAMS,
  )
  def score(h_hbm, src2d, dst2d, out_hbm, sring, dring, *rest):
    sbufs = rest[:nb]
    dbufs = rest[nb:2 * nb]
    out_all = rest[2 * nb]
    gsems = rest[2 * nb + 1:3 * nb + 1]
    gsemd = rest[3 * nb + 1:4 * nb + 1]
    isems = rest[4 * nb + 1:5 * nb + 1]
    isemd = rest[5 * nb + 1:]
    c = lax.axis_index("c")
    s = lax.axis_index("s")
    wid = c * NS + s
    cbase = wid * nchunk
    iot = lax.iota(jnp.int32, 16)

    for b in range(nb):
      pltpu.sync_copy(src2d.at[cbase + b], sring.at[b])
      pltpu.sync_copy(dst2d.at[cbase + b], dring.at[b])
      pltpu.async_copy(h_hbm.at[sring.at[b]], sbufs[b], gsems[b])
      pltpu.async_copy(h_hbm.at[dring.at[b]], dbufs[b], gsemd[b])

    @pl.loop(0, n_outer)
    def _(o):
      for b in range(nb):
        i = o * nb + b
        b1 = (b + 1) % nb

        @pl.when(i < nchunk)
        def _(i=i, b=b, b1=b1):
          pltpu.make_async_copy(h_hbm.at[sring.at[b]], sbufs[b],
                                gsems[b]).wait()
          pltpu.make_async_copy(h_hbm.at[dring.at[b]], dbufs[b],
                                gsemd[b]).wait()
          j = i + 1

          @pl.when(jnp.logical_and(j >= nb, j < nchunk))
          def _():
            pltpu.make_async_copy(src2d.at[cbase + j], sring.at[b1],
                                  isems[b1]).wait()
            pltpu.make_async_copy(dst2d.at[cbase + j], dring.at[b1],
                                  isemd[b1]).wait()
            pltpu.async_copy(h_hbm.at[sring.at[b1]], sbufs[b1], gsems[b1])
            pltpu.async_copy(h_hbm.at[dring.at[b1]], dbufs[b1], gsemd[b1])

          @plsc.parallel_loop(0, CH // 16)
          def _(g):
            score_vec = jnp.zeros((16,), jnp.float32)
            for el in range(16):
              e = g * 16 + el
              acc = []
              for k in range(width // 16):
                sv = sbufs[b][e, pl.ds(k * 16, 16)]
                dv = dbufs[b][e, pl.ds(k * 16, 16)]
                acc.append(sv * dv)
              while len(acc) > 1:
                acc = [a + c for a, c in zip(acc[::2], acc[1::2])]
              sc = jnp.sum(acc[0])
              score_vec = jnp.where(iot == el, jnp.broadcast_to(sc, (16,)),
                                    score_vec)
            out_all[i, pl.ds(g * 16, 16)] = score_vec

          @pl.when(i + nb < nchunk)
          def _():
            pltpu.async_copy(src2d.at[cbase + i + nb], sring.at[b], isems[b])
            pltpu.async_copy(dst2d.at[cbase + i + nb], dring.at[b], isemd[b])

    pltpu.sync_copy(out_all, out_hbm.at[pl.ds(cbase, nchunk)])

  return score


def _tc1(x, wn1):
  n, din = x.shape
  blk = 1000

  def body(x_ref, w_ref, o_ref):
    y = jnp.dot(x_ref[...], w_ref[...], preferred_element_type=jnp.float32)
    pad_iota = lax.broadcasted_iota(jnp.int32, (blk, 16), 1)
    extra = jnp.where(pad_iota == 0, 1.0, 0.0).astype(jnp.float32)
    o_ref[...] = jnp.concatenate([y, extra], axis=1)

  return pl.pallas_call(
      body,
      grid=(n // blk,),
      in_specs=[
          pl.BlockSpec((blk, din), lambda i: (i, 0)),
          pl.BlockSpec((din, 128), lambda i: (0, 0)),
      ],
      out_specs=pl.BlockSpec((blk, 144), lambda i: (i, 0)),
      out_shape=jax.ShapeDtypeStruct((n, 144), jnp.float32),
  )(x, wn1)


def _tc2(x, ws1, b1, wn2, agg1):
  n, din = x.shape
  blk = 1000

  def body(x_ref, ws_ref, b_ref, wn2_ref, agg_ref, h_ref, y2_ref, inv_ref):
    agg = agg_ref[0] + agg_ref[1]
    deg = agg[:, 128:129]
    inv = 1.0 / jnp.maximum(deg, 1.0)
    mean = agg[:, :128] * inv
    h = jnp.dot(x_ref[...], ws_ref[...], preferred_element_type=jnp.float32)
    h = jnp.maximum(h + mean + b_ref[...], 0.0)
    h_ref[...] = h
    y2_ref[...] = jnp.dot(h, wn2_ref[...], preferred_element_type=jnp.float32)
    inv_ref[...] = jnp.broadcast_to(inv, (blk, 128))

  return pl.pallas_call(
      body,
      grid=(n // blk,),
      in_specs=[
          pl.BlockSpec((blk, din), lambda i: (i, 0)),
          pl.BlockSpec((din, 128), lambda i: (0, 0)),
          pl.BlockSpec((1, 128), lambda i: (0, 0)),
          pl.BlockSpec((128, 128), lambda i: (0, 0)),
          pl.BlockSpec((2, blk, 144), lambda i: (0, i, 0)),
      ],
      out_specs=[
          pl.BlockSpec((blk, 128), lambda i: (i, 0)),
          pl.BlockSpec((blk, 128), lambda i: (i, 0)),
          pl.BlockSpec((blk, 128), lambda i: (i, 0)),
      ],
      out_shape=[
          jax.ShapeDtypeStruct((n, 128), jnp.float32),
          jax.ShapeDtypeStruct((n, 128), jnp.float32),
          jax.ShapeDtypeStruct((n, 128), jnp.float32),
      ],
  )(x, ws1, b1, wn2, agg1)


def _tc3(h1, ws2, b2, agg2, inv):
  n, din = h1.shape
  blk = 1000

  def body(h1_ref, ws_ref, b_ref, agg_ref, inv_ref, o_ref):
    agg = agg_ref[0] + agg_ref[1]
    h = jnp.dot(h1_ref[...], ws_ref[...], preferred_element_type=jnp.float32)
    o_ref[...] = h + agg * inv_ref[...] + b_ref[...]

  return pl.pallas_call(
      body,
      grid=(n // blk,),
      in_specs=[
          pl.BlockSpec((blk, din), lambda i: (i, 0)),
          pl.BlockSpec((din, 128), lambda i: (0, 0)),
          pl.BlockSpec((1, 128), lambda i: (0, 0)),
          pl.BlockSpec((2, blk, 128), lambda i: (0, i, 0)),
          pl.BlockSpec((blk, 128), lambda i: (i, 0)),
      ],
      out_specs=pl.BlockSpec((blk, 128), lambda i: (i, 0)),
      out_shape=jax.ShapeDtypeStruct((n, 128), jnp.float32),
  )(h1, ws2, b2, agg2, inv)


def kernel(features, edge_index, W_self1, W_neigh1, b1, W_self2, W_neigh2,
           b2):
  n, _ = features.shape
  n_edges = edge_index.shape[1]
  src = edge_index[0].astype(jnp.int32).reshape(n_edges // CH, CH)
  dst = edge_index[1].astype(jnp.int32).reshape(n_edges // CH, CH)

  segsum_ext = _make_segsum(n, n_edges, 144, nb=3)
  segsum = _make_segsum(n, n_edges, 128, nb=4)
  score_k = _make_score(n, n_edges, 128)

  y1e = _tc1(features, W_neigh1)
  agg1 = segsum_ext(y1e, src, dst).reshape(2, n, 144)
  h1, y2, inv = _tc2(features, W_self1, b1.reshape(1, 128), W_neigh2, agg1)
  agg2 = segsum(y2, src, dst).reshape(2, n, 128)
  h2 = _tc3(h1, W_self2, b2.reshape(1, 128), agg2, inv)
  return score_k(h2, src, dst).reshape(n_edges)


# speedup vs baseline: 1.0275x; 1.0066x over previous
import functools

import jax
import jax.numpy as jnp
from jax import lax
from jax.experimental import pallas as pl
from jax.experimental.pallas import tpu as pltpu
from jax.experimental.pallas import tpu_sc as plsc

NC = 2
NS = 16
NW = NC * NS

CH = 80
NB = 4


def _sc_mesh():
  return plsc.VectorSubcoreMesh(
      core_axis_name="c", subcore_axis_name="s", num_cores=NC,
      num_subcores=NS)


_SC_PARAMS = pltpu.CompilerParams(
    use_tc_tiling_on_sc=False, needs_layout_passes=False)

def _shuffle_add(v, iot):
  dnums = lax.GatherDimensionNumbers(
      offset_dims=(), collapsed_slice_dims=(0,), start_index_map=(0,))
  for sh in (8, 4, 2, 1):
    idx = jnp.bitwise_xor(iot, jnp.int32(sh)).reshape(16, 1)
    v = v + lax.gather(v, idx, dnums, slice_sizes=(1,),
                       mode=lax.GatherScatterMode.PROMISE_IN_BOUNDS)
  return v


def _zero_2d(ref, rows, width):
  zv = jnp.zeros((16,), jnp.float32)

  @pl.loop(0, rows)
  def _(r):
    for k in range(width // 16):
      ref[r, pl.ds(k * 16, 16)] = zv


def _make_segsum(n_nodes, n_edges, width, nb=3):
  ept = n_edges // NW
  nchunk = ept // CH
  n_outer = (nchunk + nb - 1) // nb
  rows_pt = n_nodes // NS
  nfull, tail = divmod(rows_pt, CH)

  scratch = (
      [pltpu.VMEM((nb, CH), jnp.int32)] * 2
      + [pltpu.VMEM((CH, width), jnp.float32) for _ in range(nb)]
      + [pltpu.VMEM_SHARED((n_nodes, width), jnp.float32)]
      + [pltpu.SemaphoreType.DMA] * (3 * nb)
  )

  @functools.partial(
      pl.kernel,
      out_type=jax.ShapeDtypeStruct((NC * n_nodes, width), jnp.float32),
      mesh=_sc_mesh(),
      scratch_types=scratch,
      compiler_params=_SC_PARAMS,
  )
  def segsum(y_hbm, src2d, dst2d, out_hbm, sring, dring, *rest):
    bufs = rest[:nb]
    agg_sh = rest[nb]
    gsem = rest[nb + 1:2 * nb + 1]
    isems = rest[2 * nb + 1:3 * nb + 1]
    isemd = rest[3 * nb + 1:]
    c = lax.axis_index("c")
    s = lax.axis_index("s")
    wid = c * NS + s
    cbase = wid * nchunk

    _zero_2d(bufs[0], CH, width)
    for r in range(nfull):
      pltpu.sync_copy(bufs[0], agg_sh.at[pl.ds(s * rows_pt + r * CH, CH)])
    if tail:
      pltpu.sync_copy(bufs[0].at[pl.ds(0, tail)],
                      agg_sh.at[pl.ds(s * rows_pt + nfull * CH, tail)])
    plsc.subcore_barrier()

    for b in range(nb):
      pltpu.sync_copy(src2d.at[cbase + b], sring.at[b])
      pltpu.sync_copy(dst2d.at[cbase + b], dring.at[b])
      pltpu.async_copy(y_hbm.at[sring.at[b]], bufs[b], gsem[b])

    @pl.loop(0, n_outer)
    def _(o):
      for b in range(nb):
        i = o * nb + b
        b1 = (b + 1) % nb

        @pl.when(i < nchunk)
        def _(i=i, b=b, b1=b1):
          pltpu.make_async_copy(y_hbm.at[sring.at[b]], bufs[b],
                                gsem[b]).wait()
          j = i + 1

          @pl.when(jnp.logical_and(j >= nb, j < nchunk))
          def _():
            pltpu.make_async_copy(src2d.at[cbase + j], sring.at[b1],
                                  isems[b1]).wait()
            pltpu.make_async_copy(dst2d.at[cbase + j], dring.at[b1],
                                  isemd[b1]).wait()
            pltpu.async_copy(y_hbm.at[sring.at[b1]], bufs[b1], gsem[b1])

          pltpu.sync_copy(bufs[b], agg_sh.at[dring.at[b]], add=True)

          @pl.when(i + nb < nchunk)
          def _():
            pltpu.async_copy(src2d.at[cbase + i + nb], sring.at[b], isems[b])
            pltpu.async_copy(dst2d.at[cbase + i + nb], dring.at[b], isemd[b])

    plsc.subcore_barrier()

    for r in range(nfull):
      row0 = s * rows_pt + r * CH
      pltpu.sync_copy(agg_sh.at[pl.ds(row0, CH)], bufs[0])
      pltpu.sync_copy(bufs[0], out_hbm.at[pl.ds(c * n_nodes + row0, CH)])
    if tail:
      row0 = s * rows_pt + nfull * CH
      pltpu.sync_copy(agg_sh.at[pl.ds(row0, tail)],
                      bufs[0].at[pl.ds(0, tail)])
      pltpu.sync_copy(bufs[0].at[pl.ds(0, tail)],
                      out_hbm.at[pl.ds(c * n_nodes + row0, tail)])

  return segsum


def _make_score(n_nodes, n_edges, width, nb=3):
  ept = n_edges // NW
  nchunk = ept // CH
  n_outer = (nchunk + nb - 1) // nb

  scratch = (
      [pltpu.VMEM((nb, CH), jnp.int32)] * 2
      + [pltpu.VMEM((CH, width), jnp.float32) for _ in range(2 * nb)]
      + [pltpu.VMEM((nchunk, CH), jnp.float32)]
      + [pltpu.SemaphoreType.DMA] * (4 * nb)
  )

  @functools.partial(
      pl.kernel,
      out_type=jax.ShapeDtypeStruct((n_edges // CH, CH), jnp.float32),
      mesh=_sc_mesh(),
      scratch_types=scratch,
      compiler_params=_SC_PARAMS,
  )
  def score(h_hbm, src2d, dst2d, out_hbm, sring, dring, *rest):
    sbufs = rest[:nb]
    dbufs = rest[nb:2 * nb]
    out_all = rest[2 * nb]
    gsems = rest[2 * nb + 1:3 * nb + 1]
    gsemd = rest[3 * nb + 1:4 * nb + 1]
    isems = rest[4 * nb + 1:5 * nb + 1]
    isemd = rest[5 * nb + 1:]
    c = lax.axis_index("c")
    s = lax.axis_index("s")
    wid = c * NS + s
    cbase = wid * nchunk
    iot = lax.iota(jnp.int32, 16)

    for b in range(nb):
      pltpu.sync_copy(src2d.at[cbase + b], sring.at[b])
      pltpu.sync_copy(dst2d.at[cbase + b], dring.at[b])
      pltpu.async_copy(h_hbm.at[sring.at[b]], sbufs[b], gsems[b])
      pltpu.async_copy(h_hbm.at[dring.at[b]], dbufs[b], gsemd[b])

    @pl.loop(0, n_outer)
    def _(o):
      for b in range(nb):
        i = o * nb + b
        b1 = (b + 1) % nb

        @pl.when(i < nchunk)
        def _(i=i, b=b, b1=b1):
          pltpu.make_async_copy(h_hbm.at[sring.at[b]], sbufs[b],
                                gsems[b]).wait()
          pltpu.make_async_copy(h_hbm.at[dring.at[b]], dbufs[b],
                                gsemd[b]).wait()
          j = i + 1

          @pl.when(jnp.logical_and(j >= nb, j < nchunk))
          def _():
            pltpu.make_async_copy(src2d.at[cbase + j], sring.at[b1],
                                  isems[b1]).wait()
            pltpu.make_async_copy(dst2d.at[cbase + j], dring.at[b1],
                                  isemd[b1]).wait()
            pltpu.async_copy(h_hbm.at[sring.at[b1]], sbufs[b1], gsems[b1])
            pltpu.async_copy(h_hbm.at[dring.at[b1]], dbufs[b1], gsemd[b1])

          @plsc.parallel_loop(0, CH // 16)
          def _(g):
            score_vec = jnp.zeros((16,), jnp.float32)
            for el in range(16):
              e = g * 16 + el
              acc = []
              for k in range(width // 16):
                sv = sbufs[b][e, pl.ds(k * 16, 16)]
                dv = dbufs[b][e, pl.ds(k * 16, 16)]
                acc.append(sv * dv)
              while len(acc) > 1:
                acc = [a + c for a, c in zip(acc[::2], acc[1::2])]
              sc = jnp.sum(acc[0])
              score_vec = jnp.where(iot == el, jnp.broadcast_to(sc, (16,)),
                                    score_vec)
            out_all[i, pl.ds(g * 16, 16)] = score_vec

          @pl.when(i + nb < nchunk)
          def _():
            pltpu.async_copy(src2d.at[cbase + i + nb], sring.at[b], isems[b])
            pltpu.async_copy(dst2d.at[cbase + i + nb], dring.at[b], isemd[b])

    pltpu.sync_copy(out_all, out_hbm.at[pl.ds(cbase, nchunk)])

  return score


def _tc1(x, wn1):
  n, din = x.shape
  blk = 1000

  def body(x_ref, w_ref, o_ref):
    y = jnp.dot(x_ref[...], w_ref[...], preferred_element_type=jnp.float32)
    pad_iota = lax.broadcasted_iota(jnp.int32, (blk, 16), 1)
    extra = jnp.where(pad_iota == 0, 1.0, 0.0).astype(jnp.float32)
    o_ref[...] = jnp.concatenate([y, extra], axis=1)

  return pl.pallas_call(
      body,
      grid=(n // blk,),
      in_specs=[
          pl.BlockSpec((blk, din), lambda i: (i, 0)),
          pl.BlockSpec((din, 128), lambda i: (0, 0)),
      ],
      out_specs=pl.BlockSpec((blk, 144), lambda i: (i, 0)),
      out_shape=jax.ShapeDtypeStruct((n, 144), jnp.float32),
  )(x, wn1)


def _tc2(x, ws1, b1, wn2, agg1):
  n, din = x.shape
  blk = 1000

  def body(x_ref, ws_ref, b_ref, wn2_ref, agg_ref, h_ref, y2_ref, inv_ref):
    agg = agg_ref[0] + agg_ref[1]
    deg = agg[:, 128:129]
    inv = 1.0 / jnp.maximum(deg, 1.0)
    mean = agg[:, :128] * inv
    h = jnp.dot(x_ref[...], ws_ref[...], preferred_element_type=jnp.float32)
    h = jnp.maximum(h + mean + b_ref[...], 0.0)
    h_ref[...] = h
    y2_ref[...] = jnp.dot(h, wn2_ref[...], preferred_element_type=jnp.float32)
    inv_ref[...] = jnp.broadcast_to(inv, (blk, 128))

  return pl.pallas_call(
      body,
      grid=(n // blk,),
      in_specs=[
          pl.BlockSpec((blk, din), lambda i: (i, 0)),
          pl.BlockSpec((din, 128), lambda i: (0, 0)),
          pl.BlockSpec((1, 128), lambda i: (0, 0)),
          pl.BlockSpec((128, 128), lambda i: (0, 0)),
          pl.BlockSpec((2, blk, 144), lambda i: (0, i, 0)),
      ],
      out_specs=[
          pl.BlockSpec((blk, 128), lambda i: (i, 0)),
          pl.BlockSpec((blk, 128), lambda i: (i, 0)),
          pl.BlockSpec((blk, 128), lambda i: (i, 0)),
      ],
      out_shape=[
          jax.ShapeDtypeStruct((n, 128), jnp.float32),
          jax.ShapeDtypeStruct((n, 128), jnp.float32),
          jax.ShapeDtypeStruct((n, 128), jnp.float32),
      ],
  )(x, ws1, b1, wn2, agg1)


def _tc3(h1, ws2, b2, agg2, inv):
  n, din = h1.shape
  blk = 1000

  def body(h1_ref, ws_ref, b_ref, agg_ref, inv_ref, o_ref):
    agg = agg_ref[0] + agg_ref[1]
    h = jnp.dot(h1_ref[...], ws_ref[...], preferred_element_type=jnp.float32)
    o_ref[...] = h + agg * inv_ref[...] + b_ref[...]

  return pl.pallas_call(
      body,
      grid=(n // blk,),
      in_specs=[
          pl.BlockSpec((blk, din), lambda i: (i, 0)),
          pl.BlockSpec((din, 128), lambda i: (0, 0)),
          pl.BlockSpec((1, 128), lambda i: (0, 0)),
          pl.BlockSpec((2, blk, 128), lambda i: (0, i, 0)),
          pl.BlockSpec((blk, 128), lambda i: (i, 0)),
      ],
      out_specs=pl.BlockSpec((blk, 128), lambda i: (i, 0)),
      out_shape=jax.ShapeDtypeStruct((n, 128), jnp.float32),
  )(h1, ws2, b2, agg2, inv)


def kernel(features, edge_index, W_self1, W_neigh1, b1, W_self2, W_neigh2,
           b2):
  n, _ = features.shape
  n_edges = edge_index.shape[1]
  src = edge_index[0].astype(jnp.int32).reshape(n_edges // CH, CH)
  dst = edge_index[1].astype(jnp.int32).reshape(n_edges // CH, CH)

  segsum_ext = _make_segsum(n, n_edges, 144, nb=3)
  segsum = _make_segsum(n, n_edges, 128, nb=3)
  score_k = _make_score(n, n_edges, 128)

  y1e = _tc1(features, W_neigh1)
  agg1 = segsum_ext(y1e, src, dst).reshape(2, n, 144)
  h1, y2, inv = _tc2(features, W_self1, b1.reshape(1, 128), W_neigh2, agg1)
  agg2 = segsum(y2, src, dst).reshape(2, n, 128)
  h2 = _tc3(h1, W_self2, b2.reshape(1, 128), agg2, inv)
  return score_k(h2, src, dst).reshape(n_edges)


# speedup vs baseline: 1.1417x; 1.1111x over previous
import functools

import jax
import jax.numpy as jnp
from jax import lax
from jax.experimental import pallas as pl
from jax.experimental.pallas import tpu as pltpu
from jax.experimental.pallas import tpu_sc as plsc

NC = 2
NS = 16
NW = NC * NS

CH = 80
NB = 4


def _sc_mesh():
  return plsc.VectorSubcoreMesh(
      core_axis_name="c", subcore_axis_name="s", num_cores=NC,
      num_subcores=NS)


_SC_PARAMS = pltpu.CompilerParams(
    use_tc_tiling_on_sc=False, needs_layout_passes=False)

def _shuffle_add(v, iot):
  dnums = lax.GatherDimensionNumbers(
      offset_dims=(), collapsed_slice_dims=(0,), start_index_map=(0,))
  for sh in (8, 4, 2, 1):
    idx = jnp.bitwise_xor(iot, jnp.int32(sh)).reshape(16, 1)
    v = v + lax.gather(v, idx, dnums, slice_sizes=(1,),
                       mode=lax.GatherScatterMode.PROMISE_IN_BOUNDS)
  return v


def _zero_2d(ref, rows, width):
  zv = jnp.zeros((16,), jnp.float32)

  @pl.loop(0, rows)
  def _(r):
    for k in range(width // 16):
      ref[r, pl.ds(k * 16, 16)] = zv


def _make_segsum(n_nodes, n_edges, width, nb=3):
  ept = n_edges // NW
  nchunk = ept // CH
  n_outer = (nchunk + nb - 1) // nb
  rows_pt = n_nodes // NS
  nfull, tail = divmod(rows_pt, CH)

  scratch = (
      [pltpu.VMEM((nb, CH), jnp.int32)] * 2
      + [pltpu.VMEM((CH, width), jnp.float32) for _ in range(nb)]
      + [pltpu.VMEM_SHARED((n_nodes, width), jnp.float32)]
      + [pltpu.SemaphoreType.DMA] * (3 * nb)
  )

  @functools.partial(
      pl.kernel,
      out_type=jax.ShapeDtypeStruct((NC * n_nodes, width), jnp.float32),
      mesh=_sc_mesh(),
      scratch_types=scratch,
      compiler_params=_SC_PARAMS,
  )
  def segsum(y_hbm, src2d, dst2d, out_hbm, sring, dring, *rest):
    bufs = rest[:nb]
    agg_sh = rest[nb]
    gsem = rest[nb + 1:2 * nb + 1]
    isems = rest[2 * nb + 1:3 * nb + 1]
    isemd = rest[3 * nb + 1:]
    c = lax.axis_index("c")
    s = lax.axis_index("s")
    wid = c * NS + s
    cbase = wid * nchunk

    _zero_2d(bufs[0], CH, width)
    for r in range(nfull):
      pltpu.sync_copy(bufs[0], agg_sh.at[pl.ds(s * rows_pt + r * CH, CH)])
    if tail:
      pltpu.sync_copy(bufs[0].at[pl.ds(0, tail)],
                      agg_sh.at[pl.ds(s * rows_pt + nfull * CH, tail)])
    plsc.subcore_barrier()

    for b in range(nb):
      pltpu.sync_copy(src2d.at[cbase + b], sring.at[b])
      pltpu.sync_copy(dst2d.at[cbase + b], dring.at[b])
      pltpu.async_copy(y_hbm.at[sring.at[b]], bufs[b], gsem[b])

    @pl.loop(0, n_outer)
    def _(o):
      for b in range(nb):
        i = o * nb + b
        b1 = (b + 1) % nb

        @pl.when(i < nchunk)
        def _(i=i, b=b, b1=b1):
          pltpu.make_async_copy(y_hbm.at[sring.at[b]], bufs[b],
                                gsem[b]).wait()
          j = i + 1

          @pl.when(jnp.logical_and(j >= nb, j < nchunk))
          def _():
            pltpu.make_async_copy(src2d.at[cbase + j], sring.at[b1],
                                  isems[b1]).wait()
            pltpu.make_async_copy(dst2d.at[cbase + j], dring.at[b1],
                                  isemd[b1]).wait()
            pltpu.async_copy(y_hbm.at[sring.at[b1]], bufs[b1], gsem[b1])

          pltpu.sync_copy(bufs[b], agg_sh.at[dring.at[b]], add=True)

          @pl.when(i + nb < nchunk)
          def _():
            pltpu.async_copy(src2d.at[cbase + i + nb], sring.at[b], isems[b])
            pltpu.async_copy(dst2d.at[cbase + i + nb], dring.at[b], isemd[b])

    plsc.subcore_barrier()

    for r in range(nfull):
      row0 = s * rows_pt + r * CH
      pltpu.sync_copy(agg_sh.at[pl.ds(row0, CH)], bufs[0])
      pltpu.sync_copy(bufs[0], out_hbm.at[pl.ds(c * n_nodes + row0, CH)])
    if tail:
      row0 = s * rows_pt + nfull * CH
      pltpu.sync_copy(agg_sh.at[pl.ds(row0, tail)],
                      bufs[0].at[pl.ds(0, tail)])
      pltpu.sync_copy(bufs[0].at[pl.ds(0, tail)],
                      out_hbm.at[pl.ds(c * n_nodes + row0, tail)])

  return segsum


def _make_score(n_nodes, n_edges, width, nb=3):
  ept = n_edges // NW
  nchunk = ept // CH
  n_outer = (nchunk + nb - 1) // nb

  scratch = (
      [pltpu.VMEM((nb, CH), jnp.int32)] * 2
      + [pltpu.VMEM((CH, width), jnp.float32) for _ in range(2 * nb)]
      + [pltpu.VMEM((nchunk, CH), jnp.float32)]
      + [pltpu.SemaphoreType.DMA] * (4 * nb)
  )

  @functools.partial(
      pl.kernel,
      out_type=jax.ShapeDtypeStruct((n_edges // CH, CH), jnp.float32),
      mesh=_sc_mesh(),
      scratch_types=scratch,
      compiler_params=_SC_PARAMS,
  )
  def score(h_hbm, src2d, dst2d, out_hbm, sring, dring, *rest):
    sbufs = rest[:nb]
    dbufs = rest[nb:2 * nb]
    out_all = rest[2 * nb]
    gsems = rest[2 * nb + 1:3 * nb + 1]
    gsemd = rest[3 * nb + 1:4 * nb + 1]
    isems = rest[4 * nb + 1:5 * nb + 1]
    isemd = rest[5 * nb + 1:]
    c = lax.axis_index("c")
    s = lax.axis_index("s")
    wid = c * NS + s
    cbase = wid * nchunk
    iot = lax.iota(jnp.int32, 16)

    for b in range(nb):
      pltpu.sync_copy(src2d.at[cbase + b], sring.at[b])
      pltpu.sync_copy(dst2d.at[cbase + b], dring.at[b])
      pltpu.async_copy(h_hbm.at[sring.at[b]], sbufs[b], gsems[b])
      pltpu.async_copy(h_hbm.at[dring.at[b]], dbufs[b], gsemd[b])

    @pl.loop(0, n_outer)
    def _(o):
      for b in range(nb):
        i = o * nb + b
        b1 = (b + 1) % nb

        @pl.when(i < nchunk)
        def _(i=i, b=b, b1=b1):
          pltpu.make_async_copy(h_hbm.at[sring.at[b]], sbufs[b],
                                gsems[b]).wait()
          pltpu.make_async_copy(h_hbm.at[dring.at[b]], dbufs[b],
                                gsemd[b]).wait()
          j = i + 1

          @pl.when(jnp.logical_and(j >= nb, j < nchunk))
          def _():
            pltpu.make_async_copy(src2d.at[cbase + j], sring.at[b1],
                                  isems[b1]).wait()
            pltpu.make_async_copy(dst2d.at[cbase + j], dring.at[b1],
                                  isemd[b1]).wait()
            pltpu.async_copy(h_hbm.at[sring.at[b1]], sbufs[b1], gsems[b1])
            pltpu.async_copy(h_hbm.at[dring.at[b1]], dbufs[b1], gsemd[b1])

          @plsc.parallel_loop(0, CH // 16)
          def _(g):
            score_vec = jnp.zeros((16,), jnp.float32)
            for el in range(16):
              e = g * 16 + el
              acc = None
              for k in range(width // 16):
                sv = sbufs[b][e, pl.ds(k * 16, 16)]
                dv = dbufs[b][e, pl.ds(k * 16, 16)]
                p = sv * dv
                acc = p if acc is None else acc + p
              sc = jnp.sum(acc)
              score_vec = jnp.where(iot == el, jnp.broadcast_to(sc, (16,)),
                                    score_vec)
            out_all[i, pl.ds(g * 16, 16)] = score_vec

          @pl.when(i + nb < nchunk)
          def _():
            pltpu.async_copy(src2d.at[cbase + i + nb], sring.at[b], isems[b])
            pltpu.async_copy(dst2d.at[cbase + i + nb], dring.at[b], isemd[b])

    pltpu.sync_copy(out_all, out_hbm.at[pl.ds(cbase, nchunk)])

  return score


def _tc1(x, wn1):
  n, din = x.shape
  blk = 1000

  def body(x_ref, w_ref, o_ref):
    y = jnp.dot(x_ref[...], w_ref[...], preferred_element_type=jnp.float32)
    pad_iota = lax.broadcasted_iota(jnp.int32, (blk, 16), 1)
    extra = jnp.where(pad_iota == 0, 1.0, 0.0).astype(jnp.float32)
    o_ref[...] = jnp.concatenate([y, extra], axis=1)

  return pl.pallas_call(
      body,
      grid=(n // blk,),
      in_specs=[
          pl.BlockSpec((blk, din), lambda i: (i, 0)),
          pl.BlockSpec((din, 128), lambda i: (0, 0)),
      ],
      out_specs=pl.BlockSpec((blk, 144), lambda i: (i, 0)),
      out_shape=jax.ShapeDtypeStruct((n, 144), jnp.float32),
  )(x, wn1)


def _tc2(x, ws1, b1, wn2, agg1):
  n, din = x.shape
  blk = 1000

  def body(x_ref, ws_ref, b_ref, wn2_ref, agg_ref, h_ref, y2_ref, inv_ref):
    agg = agg_ref[0] + agg_ref[1]
    deg = agg[:, 128:129]
    inv = 1.0 / jnp.maximum(deg, 1.0)
    mean = agg[:, :128] * inv
    h = jnp.dot(x_ref[...], ws_ref[...], preferred_element_type=jnp.float32)
    h = jnp.maximum(h + mean + b_ref[...], 0.0)
    h_ref[...] = h
    y2_ref[...] = jnp.dot(h, wn2_ref[...], preferred_element_type=jnp.float32)
    inv_ref[...] = jnp.broadcast_to(inv, (blk, 128))

  return pl.pallas_call(
      body,
      grid=(n // blk,),
      in_specs=[
          pl.BlockSpec((blk, din), lambda i: (i, 0)),
          pl.BlockSpec((din, 128), lambda i: (0, 0)),
          pl.BlockSpec((1, 128), lambda i: (0, 0)),
          pl.BlockSpec((128, 128), lambda i: (0, 0)),
          pl.BlockSpec((2, blk, 144), lambda i: (0, i, 0)),
      ],
      out_specs=[
          pl.BlockSpec((blk, 128), lambda i: (i, 0)),
          pl.BlockSpec((blk, 128), lambda i: (i, 0)),
          pl.BlockSpec((blk, 128), lambda i: (i, 0)),
      ],
      out_shape=[
          jax.ShapeDtypeStruct((n, 128), jnp.float32),
          jax.ShapeDtypeStruct((n, 128), jnp.float32),
          jax.ShapeDtypeStruct((n, 128), jnp.float32),
      ],
  )(x, ws1, b1, wn2, agg1)


def _tc3(h1, ws2, b2, agg2, inv):
  n, din = h1.shape
  blk = 1000

  def body(h1_ref, ws_ref, b_ref, agg_ref, inv_ref, o_ref):
    agg = agg_ref[0] + agg_ref[1]
    h = jnp.dot(h1_ref[...], ws_ref[...], preferred_element_type=jnp.float32)
    o_ref[...] = h + agg * inv_ref[...] + b_ref[...]

  return pl.pallas_call(
      body,
      grid=(n // blk,),
      in_specs=[
          pl.BlockSpec((blk, din), lambda i: (i, 0)),
          pl.BlockSpec((din, 128), lambda i: (0, 0)),
          pl.BlockSpec((1, 128), lambda i: (0, 0)),
          pl.BlockSpec((2, blk, 128), lambda i: (0, i, 0)),
          pl.BlockSpec((blk, 128), lambda i: (i, 0)),
      ],
      out_specs=pl.BlockSpec((blk, 128), lambda i: (i, 0)),
      out_shape=jax.ShapeDtypeStruct((n, 128), jnp.float32),
  )(h1, ws2, b2, agg2, inv)


def kernel(features, edge_index, W_self1, W_neigh1, b1, W_self2, W_neigh2,
           b2):
  n, _ = features.shape
  n_edges = edge_index.shape[1]
  src = edge_index[0].astype(jnp.int32).reshape(n_edges // CH, CH)
  dst = edge_index[1].astype(jnp.int32).reshape(n_edges // CH, CH)

  segsum_ext = _make_segsum(n, n_edges, 144, nb=3)
  segsum = _make_segsum(n, n_edges, 128, nb=3)
  score_k = _make_score(n, n_edges, 128)

  y1e = _tc1(features, W_neigh1)
  agg1 = segsum_ext(y1e, src, dst).reshape(2, n, 144)
  h1, y2, inv = _tc2(features, W_self1, b1.reshape(1, 128), W_neigh2, agg1)
  agg2 = segsum(y2, src, dst).reshape(2, n, 128)
  h2 = _tc3(h1, W_self2, b2.reshape(1, 128), agg2, inv)
  return score_k(h2, src, dst).reshape(n_edges)


# speedup vs baseline: 1.1442x; 1.0022x over previous
import functools

import jax
import jax.numpy as jnp
from jax import lax
from jax.experimental import pallas as pl
from jax.experimental.pallas import tpu as pltpu
from jax.experimental.pallas import tpu_sc as plsc

NC = 2
NS = 16
NW = NC * NS

CH = 80
NB = 4


def _sc_mesh():
  return plsc.VectorSubcoreMesh(
      core_axis_name="c", subcore_axis_name="s", num_cores=NC,
      num_subcores=NS)


_SC_PARAMS = pltpu.CompilerParams(
    use_tc_tiling_on_sc=False, needs_layout_passes=False)

def _shuffle_add(v, iot):
  dnums = lax.GatherDimensionNumbers(
      offset_dims=(), collapsed_slice_dims=(0,), start_index_map=(0,))
  for sh in (8, 4, 2, 1):
    idx = jnp.bitwise_xor(iot, jnp.int32(sh)).reshape(16, 1)
    v = v + lax.gather(v, idx, dnums, slice_sizes=(1,),
                       mode=lax.GatherScatterMode.PROMISE_IN_BOUNDS)
  return v


def _zero_2d(ref, rows, width):
  zv = jnp.zeros((16,), jnp.float32)

  @pl.loop(0, rows)
  def _(r):
    for k in range(width // 16):
      ref[r, pl.ds(k * 16, 16)] = zv


def _make_segsum(n_nodes, n_edges, width, nb=3):
  ept = n_edges // NW
  nchunk = ept // CH
  n_outer = (nchunk + nb - 1) // nb
  rows_pt = n_nodes // NS
  nfull, tail = divmod(rows_pt, CH)

  scratch = (
      [pltpu.VMEM((nb, CH), jnp.int32)] * 2
      + [pltpu.VMEM((CH, width), jnp.float32) for _ in range(nb)]
      + [pltpu.VMEM_SHARED((n_nodes, width), jnp.float32)]
      + [pltpu.SemaphoreType.DMA] * (3 * nb)
  )

  @functools.partial(
      pl.kernel,
      out_type=jax.ShapeDtypeStruct((NC * n_nodes, width), jnp.float32),
      mesh=_sc_mesh(),
      scratch_types=scratch,
      compiler_params=_SC_PARAMS,
  )
  def segsum(y_hbm, src2d, dst2d, out_hbm, sring, dring, *rest):
    bufs = rest[:nb]
    agg_sh = rest[nb]
    gsem = rest[nb + 1:2 * nb + 1]
    isems = rest[2 * nb + 1:3 * nb + 1]
    isemd = rest[3 * nb + 1:]
    c = lax.axis_index("c")
    s = lax.axis_index("s")
    wid = c * NS + s
    cbase = wid * nchunk

    _zero_2d(bufs[0], CH, width)
    for r in range(nfull):
      pltpu.sync_copy(bufs[0], agg_sh.at[pl.ds(s * rows_pt + r * CH, CH)])
    if tail:
      pltpu.sync_copy(bufs[0].at[pl.ds(0, tail)],
                      agg_sh.at[pl.ds(s * rows_pt + nfull * CH, tail)])
    plsc.subcore_barrier()

    for b in range(nb):
      pltpu.sync_copy(src2d.at[cbase + b], sring.at[b])
      pltpu.sync_copy(dst2d.at[cbase + b], dring.at[b])
      pltpu.async_copy(y_hbm.at[sring.at[b]], bufs[b], gsem[b])

    @pl.loop(0, n_outer)
    def _(o):
      for b in range(nb):
        i = o * nb + b
        b1 = (b + 1) % nb

        @pl.when(i < nchunk)
        def _(i=i, b=b, b1=b1):
          pltpu.make_async_copy(y_hbm.at[sring.at[b]], bufs[b],
                                gsem[b]).wait()
          j = i + 1

          @pl.when(jnp.logical_and(j >= nb, j < nchunk))
          def _():
            pltpu.make_async_copy(src2d.at[cbase + j], sring.at[b1],
                                  isems[b1]).wait()
            pltpu.make_async_copy(dst2d.at[cbase + j], dring.at[b1],
                                  isemd[b1]).wait()
            pltpu.async_copy(y_hbm.at[sring.at[b1]], bufs[b1], gsem[b1])

          pltpu.sync_copy(bufs[b], agg_sh.at[dring.at[b]], add=True)

          @pl.when(i + nb < nchunk)
          def _():
            pltpu.async_copy(src2d.at[cbase + i + nb], sring.at[b], isems[b])
            pltpu.async_copy(dst2d.at[cbase + i + nb], dring.at[b], isemd[b])

    plsc.subcore_barrier()

    for r in range(nfull):
      row0 = s * rows_pt + r * CH
      pltpu.sync_copy(agg_sh.at[pl.ds(row0, CH)], bufs[0])
      pltpu.sync_copy(bufs[0], out_hbm.at[pl.ds(c * n_nodes + row0, CH)])
    if tail:
      row0 = s * rows_pt + nfull * CH
      pltpu.sync_copy(agg_sh.at[pl.ds(row0, tail)],
                      bufs[0].at[pl.ds(0, tail)])
      pltpu.sync_copy(bufs[0].at[pl.ds(0, tail)],
                      out_hbm.at[pl.ds(c * n_nodes + row0, tail)])

  return segsum


def _make_score(n_nodes, n_edges, width, nb=3):
  ept = n_edges // NW
  nchunk = ept // CH
  n_outer = (nchunk + nb - 1) // nb

  scratch = (
      [pltpu.VMEM((nb, CH), jnp.int32)] * 2
      + [pltpu.VMEM((CH, width), jnp.bfloat16) for _ in range(2 * nb)]
      + [pltpu.VMEM((nchunk, CH), jnp.float32)]
      + [pltpu.SemaphoreType.DMA] * (4 * nb)
  )

  @functools.partial(
      pl.kernel,
      out_type=jax.ShapeDtypeStruct((n_edges // CH, CH), jnp.float32),
      mesh=_sc_mesh(),
      scratch_types=scratch,
      compiler_params=_SC_PARAMS,
  )
  def score(h_hbm, src2d, dst2d, out_hbm, sring, dring, *rest):
    sbufs = rest[:nb]
    dbufs = rest[nb:2 * nb]
    out_all = rest[2 * nb]
    gsems = rest[2 * nb + 1:3 * nb + 1]
    gsemd = rest[3 * nb + 1:4 * nb + 1]
    isems = rest[4 * nb + 1:5 * nb + 1]
    isemd = rest[5 * nb + 1:]
    c = lax.axis_index("c")
    s = lax.axis_index("s")
    wid = c * NS + s
    cbase = wid * nchunk
    iot = lax.iota(jnp.int32, 16)

    for b in range(nb):
      pltpu.sync_copy(src2d.at[cbase + b], sring.at[b])
      pltpu.sync_copy(dst2d.at[cbase + b], dring.at[b])
      pltpu.async_copy(h_hbm.at[sring.at[b]], sbufs[b], gsems[b])
      pltpu.async_copy(h_hbm.at[dring.at[b]], dbufs[b], gsemd[b])

    @pl.loop(0, n_outer)
    def _(o):
      for b in range(nb):
        i = o * nb + b
        b1 = (b + 1) % nb

        @pl.when(i < nchunk)
        def _(i=i, b=b, b1=b1):
          pltpu.make_async_copy(h_hbm.at[sring.at[b]], sbufs[b],
                                gsems[b]).wait()
          pltpu.make_async_copy(h_hbm.at[dring.at[b]], dbufs[b],
                                gsemd[b]).wait()
          j = i + 1

          @pl.when(jnp.logical_and(j >= nb, j < nchunk))
          def _():
            pltpu.make_async_copy(src2d.at[cbase + j], sring.at[b1],
                                  isems[b1]).wait()
            pltpu.make_async_copy(dst2d.at[cbase + j], dring.at[b1],
                                  isemd[b1]).wait()
            pltpu.async_copy(h_hbm.at[sring.at[b1]], sbufs[b1], gsems[b1])
            pltpu.async_copy(h_hbm.at[dring.at[b1]], dbufs[b1], gsemd[b1])

          @plsc.parallel_loop(0, CH // 16)
          def _(g):
            score_vec = jnp.zeros((16,), jnp.float32)
            for el in range(16):
              e = g * 16 + el
              acc = None
              for k in range(width // 32):
                sv = sbufs[b][e, pl.ds(k * 32, 32)]
                dv = dbufs[b][e, pl.ds(k * 32, 32)]
                s0, s1 = plsc.unpack(sv, format=plsc.PackFormat.INTERLEAVED)
                d0, d1 = plsc.unpack(dv, format=plsc.PackFormat.INTERLEAVED)
                p = s0 * d0 + s1 * d1
                acc = p if acc is None else acc + p
              sc = jnp.sum(acc)
              score_vec = jnp.where(iot == el, jnp.broadcast_to(sc, (16,)),
                                    score_vec)
            out_all[i, pl.ds(g * 16, 16)] = score_vec

          @pl.when(i + nb < nchunk)
          def _():
            pltpu.async_copy(src2d.at[cbase + i + nb], sring.at[b], isems[b])
            pltpu.async_copy(dst2d.at[cbase + i + nb], dring.at[b], isemd[b])

    pltpu.sync_copy(out_all, out_hbm.at[pl.ds(cbase, nchunk)])

  return score


def _tc1(x, wn1):
  n, din = x.shape
  blk = 1000

  def body(x_ref, w_ref, o_ref):
    y = jnp.dot(x_ref[...], w_ref[...], preferred_element_type=jnp.float32)
    pad_iota = lax.broadcasted_iota(jnp.int32, (blk, 16), 1)
    extra = jnp.where(pad_iota == 0, 1.0, 0.0).astype(jnp.float32)
    o_ref[...] = jnp.concatenate([y, extra], axis=1)

  return pl.pallas_call(
      body,
      grid=(n // blk,),
      in_specs=[
          pl.BlockSpec((blk, din), lambda i: (i, 0)),
          pl.BlockSpec((din, 128), lambda i: (0, 0)),
      ],
      out_specs=pl.BlockSpec((blk, 144), lambda i: (i, 0)),
      out_shape=jax.ShapeDtypeStruct((n, 144), jnp.float32),
  )(x, wn1)


def _tc2(x, ws1, b1, wn2, agg1):
  n, din = x.shape
  blk = 1000

  def body(x_ref, ws_ref, b_ref, wn2_ref, agg_ref, h_ref, y2_ref, inv_ref):
    agg = agg_ref[0] + agg_ref[1]
    deg = agg[:, 128:129]
    inv = 1.0 / jnp.maximum(deg, 1.0)
    mean = agg[:, :128] * inv
    h = jnp.dot(x_ref[...], ws_ref[...], preferred_element_type=jnp.float32)
    h = jnp.maximum(h + mean + b_ref[...], 0.0)
    h_ref[...] = h
    y2_ref[...] = jnp.dot(h, wn2_ref[...], preferred_element_type=jnp.float32)
    inv_ref[...] = jnp.broadcast_to(inv, (blk, 128))

  return pl.pallas_call(
      body,
      grid=(n // blk,),
      in_specs=[
          pl.BlockSpec((blk, din), lambda i: (i, 0)),
          pl.BlockSpec((din, 128), lambda i: (0, 0)),
          pl.BlockSpec((1, 128), lambda i: (0, 0)),
          pl.BlockSpec((128, 128), lambda i: (0, 0)),
          pl.BlockSpec((2, blk, 144), lambda i: (0, i, 0)),
      ],
      out_specs=[
          pl.BlockSpec((blk, 128), lambda i: (i, 0)),
          pl.BlockSpec((blk, 128), lambda i: (i, 0)),
          pl.BlockSpec((blk, 128), lambda i: (i, 0)),
      ],
      out_shape=[
          jax.ShapeDtypeStruct((n, 128), jnp.float32),
          jax.ShapeDtypeStruct((n, 128), jnp.float32),
          jax.ShapeDtypeStruct((n, 128), jnp.float32),
      ],
  )(x, ws1, b1, wn2, agg1)


def _tc3(h1, ws2, b2, agg2, inv):
  n, din = h1.shape
  blk = 1000

  def body(h1_ref, ws_ref, b_ref, agg_ref, inv_ref, o_ref):
    agg = agg_ref[0] + agg_ref[1]
    h = jnp.dot(h1_ref[...], ws_ref[...], preferred_element_type=jnp.float32)
    o_ref[...] = (h + agg * inv_ref[...] + b_ref[...]).astype(jnp.bfloat16)

  return pl.pallas_call(
      body,
      grid=(n // blk,),
      in_specs=[
          pl.BlockSpec((blk, din), lambda i: (i, 0)),
          pl.BlockSpec((din, 128), lambda i: (0, 0)),
          pl.BlockSpec((1, 128), lambda i: (0, 0)),
          pl.BlockSpec((2, blk, 128), lambda i: (0, i, 0)),
          pl.BlockSpec((blk, 128), lambda i: (i, 0)),
      ],
      out_specs=pl.BlockSpec((blk, 128), lambda i: (i, 0)),
      out_shape=jax.ShapeDtypeStruct((n, 128), jnp.bfloat16),
  )(h1, ws2, b2, agg2, inv)


def kernel(features, edge_index, W_self1, W_neigh1, b1, W_self2, W_neigh2,
           b2):
  n, _ = features.shape
  n_edges = edge_index.shape[1]
  src = edge_index[0].astype(jnp.int32).reshape(n_edges // CH, CH)
  dst = edge_index[1].astype(jnp.int32).reshape(n_edges // CH, CH)

  segsum_ext = _make_segsum(n, n_edges, 144, nb=3)
  segsum = _make_segsum(n, n_edges, 128, nb=3)
  score_k = _make_score(n, n_edges, 128)

  y1e = _tc1(features, W_neigh1)
  agg1 = segsum_ext(y1e, src, dst).reshape(2, n, 144)
  h1, y2, inv = _tc2(features, W_self1, b1.reshape(1, 128), W_neigh2, agg1)
  agg2 = segsum(y2, src, dst).reshape(2, n, 128)
  h2 = _tc3(h1, W_self2, b2.reshape(1, 128), agg2, inv)
  return score_k(h2, src, dst).reshape(n_edges)


# speedup vs baseline: 1.4050x; 1.2279x over previous
import functools

import jax
import jax.numpy as jnp
from jax import lax
from jax.experimental import pallas as pl
from jax.experimental.pallas import tpu as pltpu
from jax.experimental.pallas import tpu_sc as plsc

NC = 2
NS = 16
NW = NC * NS

CH = 80
NB = 4


def _sc_mesh():
  return plsc.VectorSubcoreMesh(
      core_axis_name="c", subcore_axis_name="s", num_cores=NC,
      num_subcores=NS)


_SC_PARAMS = pltpu.CompilerParams(
    use_tc_tiling_on_sc=False, needs_layout_passes=False)

def _shuffle_add(v, iot):
  dnums = lax.GatherDimensionNumbers(
      offset_dims=(), collapsed_slice_dims=(0,), start_index_map=(0,))
  for sh in (8, 4, 2, 1):
    idx = jnp.bitwise_xor(iot, jnp.int32(sh)).reshape(16, 1)
    v = v + lax.gather(v, idx, dnums, slice_sizes=(1,),
                       mode=lax.GatherScatterMode.PROMISE_IN_BOUNDS)
  return v


def _zero_2d(ref, rows, width):
  zv = jnp.zeros((16,), jnp.float32)

  @pl.loop(0, rows)
  def _(r):
    for k in range(width // 16):
      ref[r, pl.ds(k * 16, 16)] = zv


def _make_segsum(n_nodes, n_edges, width, nb=2):
  ept = n_edges // NW
  nchunk = ept // CH
  n_outer = (nchunk + nb - 1) // nb
  rows_pt = n_nodes // NS
  nfull, tail = divmod(rows_pt, CH)

  scratch = (
      [pltpu.VMEM((nb, CH), jnp.int32)] * 2
      + [pltpu.VMEM((CH, width), jnp.float32) for _ in range(nb)]
      + [pltpu.VMEM_SHARED((n_nodes, width), jnp.float32)]
      + [pltpu.SemaphoreType.DMA] * (3 * nb)
  )

  @functools.partial(
      pl.kernel,
      out_type=jax.ShapeDtypeStruct((NC * n_nodes, width), jnp.float32),
      mesh=_sc_mesh(),
      scratch_types=scratch,
      compiler_params=_SC_PARAMS,
  )
  def segsum(y_hbm, src2d, dst2d, out_hbm, sring, dring, *rest):
    bufs = rest[:nb]
    agg_sh = rest[nb]
    gsem = rest[nb + 1:2 * nb + 1]
    isems = rest[2 * nb + 1:3 * nb + 1]
    isemd = rest[3 * nb + 1:]
    c = lax.axis_index("c")
    s = lax.axis_index("s")
    wid = c * NS + s
    cbase = wid * nchunk

    _zero_2d(bufs[0], CH, width)
    for r in range(nfull):
      pltpu.sync_copy(bufs[0], agg_sh.at[pl.ds(s * rows_pt + r * CH, CH)])
    if tail:
      pltpu.sync_copy(bufs[0].at[pl.ds(0, tail)],
                      agg_sh.at[pl.ds(s * rows_pt + nfull * CH, tail)])
    plsc.subcore_barrier()

    for b in range(nb):
      pltpu.sync_copy(src2d.at[cbase + b], sring.at[b])
      pltpu.sync_copy(dst2d.at[cbase + b], dring.at[b])
      pltpu.async_copy(y_hbm.at[sring.at[b]], bufs[b], gsem[b])

    @pl.loop(0, n_outer)
    def _(o):
      for b in range(nb):
        i = o * nb + b
        b1 = (b + 1) % nb

        @pl.when(i < nchunk)
        def _(i=i, b=b, b1=b1):
          pltpu.make_async_copy(y_hbm.at[sring.at[b]], bufs[b],
                                gsem[b]).wait()
          j = i + 1

          @pl.when(jnp.logical_and(j >= nb, j < nchunk))
          def _():
            pltpu.make_async_copy(src2d.at[cbase + j], sring.at[b1],
                                  isems[b1]).wait()
            pltpu.make_async_copy(dst2d.at[cbase + j], dring.at[b1],
                                  isemd[b1]).wait()
            pltpu.async_copy(y_hbm.at[sring.at[b1]], bufs[b1], gsem[b1])

          pltpu.sync_copy(bufs[b], agg_sh.at[dring.at[b]], add=True)

          @pl.when(i + nb < nchunk)
          def _():
            pltpu.async_copy(src2d.at[cbase + i + nb], sring.at[b], isems[b])
            pltpu.async_copy(dst2d.at[cbase + i + nb], dring.at[b], isemd[b])

    plsc.subcore_barrier()

    for r in range(nfull):
      row0 = s * rows_pt + r * CH
      pltpu.sync_copy(agg_sh.at[pl.ds(row0, CH)], bufs[0])
      pltpu.sync_copy(bufs[0], out_hbm.at[pl.ds(c * n_nodes + row0, CH)])
    if tail:
      row0 = s * rows_pt + nfull * CH
      pltpu.sync_copy(agg_sh.at[pl.ds(row0, tail)],
                      bufs[0].at[pl.ds(0, tail)])
      pltpu.sync_copy(bufs[0].at[pl.ds(0, tail)],
                      out_hbm.at[pl.ds(c * n_nodes + row0, tail)])

  return segsum


def _make_score(n_nodes, n_edges, width, nb=2):
  ept = n_edges // NW
  nchunk = ept // CH
  n_outer = (nchunk + nb - 1) // nb

  scratch = (
      [pltpu.VMEM((nb, CH), jnp.int32)] * 2
      + [pltpu.VMEM((CH, width), jnp.bfloat16) for _ in range(2 * nb)]
      + [pltpu.VMEM((nchunk, CH), jnp.float32),
         pltpu.VMEM((CH // 16, 16, 17), jnp.float32)]
      + [pltpu.SemaphoreType.DMA] * (4 * nb)
  )

  @functools.partial(
      pl.kernel,
      out_type=jax.ShapeDtypeStruct((n_edges // CH, CH), jnp.float32),
      mesh=_sc_mesh(),
      scratch_types=scratch,
      compiler_params=_SC_PARAMS,
  )
  def score(h_hbm, src2d, dst2d, out_hbm, sring, dring, *rest):
    sbufs = rest[:nb]
    dbufs = rest[nb:2 * nb]
    out_all = rest[2 * nb]
    tmp = rest[2 * nb + 1]
    gsems = rest[2 * nb + 2:3 * nb + 2]
    gsemd = rest[3 * nb + 2:4 * nb + 2]
    isems = rest[4 * nb + 2:5 * nb + 2]
    isemd = rest[5 * nb + 2:]
    c = lax.axis_index("c")
    s = lax.axis_index("s")
    wid = c * NS + s
    cbase = wid * nchunk
    iot = lax.iota(jnp.int32, 16)

    for b in range(nb):
      pltpu.sync_copy(src2d.at[cbase + b], sring.at[b])
      pltpu.sync_copy(dst2d.at[cbase + b], dring.at[b])
      pltpu.async_copy(h_hbm.at[sring.at[b]], sbufs[b], gsems[b])
      pltpu.async_copy(h_hbm.at[dring.at[b]], dbufs[b], gsemd[b])

    @pl.loop(0, n_outer)
    def _(o):
      for b in range(nb):
        i = o * nb + b
        b1 = (b + 1) % nb

        @pl.when(i < nchunk)
        def _(i=i, b=b, b1=b1):
          pltpu.make_async_copy(h_hbm.at[sring.at[b]], sbufs[b],
                                gsems[b]).wait()
          pltpu.make_async_copy(h_hbm.at[dring.at[b]], dbufs[b],
                                gsemd[b]).wait()
          j = i + 1

          @pl.when(jnp.logical_and(j >= nb, j < nchunk))
          def _():
            pltpu.make_async_copy(src2d.at[cbase + j], sring.at[b1],
                                  isems[b1]).wait()
            pltpu.make_async_copy(dst2d.at[cbase + j], dring.at[b1],
                                  isemd[b1]).wait()
            pltpu.async_copy(h_hbm.at[sring.at[b1]], sbufs[b1], gsems[b1])
            pltpu.async_copy(h_hbm.at[dring.at[b1]], dbufs[b1], gsemd[b1])

          @plsc.parallel_loop(0, CH // 16)
          def _(g):
            for el in range(16):
              e = g * 16 + el
              acc = None
              for k in range(width // 32):
                sv = sbufs[b][e, pl.ds(k * 32, 32)]
                dv = dbufs[b][e, pl.ds(k * 32, 32)]
                s0, s1 = plsc.unpack(sv, format=plsc.PackFormat.INTERLEAVED)
                d0, d1 = plsc.unpack(dv, format=plsc.PackFormat.INTERLEAVED)
                p = s0 * d0 + s1 * d1
                acc = p if acc is None else acc + p
              tmp[g, el, pl.ds(0, 16)] = acc
            gv = jnp.full((16,), g, dtype=jnp.int32)
            col = None
            for l in range(16):
              lv = jnp.full((16,), l, dtype=jnp.int32)
              cv = plsc.load_gather(tmp, [gv, iot, lv])
              col = cv if col is None else col + cv
            out_all[i, pl.ds(g * 16, 16)] = col

          @pl.when(i + nb < nchunk)
          def _():
            pltpu.async_copy(src2d.at[cbase + i + nb], sring.at[b], isems[b])
            pltpu.async_copy(dst2d.at[cbase + i + nb], dring.at[b], isemd[b])

    pltpu.sync_copy(out_all, out_hbm.at[pl.ds(cbase, nchunk)])

  return score


def _tc1(x, wn1):
  n, din = x.shape
  blk = 1000

  def body(x_ref, w_ref, o_ref):
    y = jnp.dot(x_ref[...], w_ref[...], preferred_element_type=jnp.float32)
    pad_iota = lax.broadcasted_iota(jnp.int32, (blk, 16), 1)
    extra = jnp.where(pad_iota == 0, 1.0, 0.0).astype(jnp.float32)
    o_ref[...] = jnp.concatenate([y, extra], axis=1)

  return pl.pallas_call(
      body,
      grid=(n // blk,),
      in_specs=[
          pl.BlockSpec((blk, din), lambda i: (i, 0)),
          pl.BlockSpec((din, 128), lambda i: (0, 0)),
      ],
      out_specs=pl.BlockSpec((blk, 144), lambda i: (i, 0)),
      out_shape=jax.ShapeDtypeStruct((n, 144), jnp.float32),
  )(x, wn1)


def _tc2(x, ws1, b1, wn2, agg1):
  n, din = x.shape
  blk = 1000

  def body(x_ref, ws_ref, b_ref, wn2_ref, agg_ref, h_ref, y2_ref, inv_ref):
    agg = agg_ref[0] + agg_ref[1]
    deg = agg[:, 128:129]
    inv = 1.0 / jnp.maximum(deg, 1.0)
    mean = agg[:, :128] * inv
    h = jnp.dot(x_ref[...], ws_ref[...], preferred_element_type=jnp.float32)
    h = jnp.maximum(h + mean + b_ref[...], 0.0)
    h_ref[...] = h
    y2_ref[...] = jnp.dot(h, wn2_ref[...], preferred_element_type=jnp.float32)
    inv_ref[...] = jnp.broadcast_to(inv, (blk, 128))

  return pl.pallas_call(
      body,
      grid=(n // blk,),
      in_specs=[
          pl.BlockSpec((blk, din), lambda i: (i, 0)),
          pl.BlockSpec((din, 128), lambda i: (0, 0)),
          pl.BlockSpec((1, 128), lambda i: (0, 0)),
          pl.BlockSpec((128, 128), lambda i: (0, 0)),
          pl.BlockSpec((2, blk, 144), lambda i: (0, i, 0)),
      ],
      out_specs=[
          pl.BlockSpec((blk, 128), lambda i: (i, 0)),
          pl.BlockSpec((blk, 128), lambda i: (i, 0)),
          pl.BlockSpec((blk, 128), lambda i: (i, 0)),
      ],
      out_shape=[
          jax.ShapeDtypeStruct((n, 128), jnp.float32),
          jax.ShapeDtypeStruct((n, 128), jnp.float32),
          jax.ShapeDtypeStruct((n, 128), jnp.float32),
      ],
  )(x, ws1, b1, wn2, agg1)


def _tc3(h1, ws2, b2, agg2, inv):
  n, din = h1.shape
  blk = 1000

  def body(h1_ref, ws_ref, b_ref, agg_ref, inv_ref, o_ref):
    agg = agg_ref[0] + agg_ref[1]
    h = jnp.dot(h1_ref[...], ws_ref[...], preferred_element_type=jnp.float32)
    o_ref[...] = (h + agg * inv_ref[...] + b_ref[...]).astype(jnp.bfloat16)

  return pl.pallas_call(
      body,
      grid=(n // blk,),
      in_specs=[
          pl.BlockSpec((blk, din), lambda i: (i, 0)),
          pl.BlockSpec((din, 128), lambda i: (0, 0)),
          pl.BlockSpec((1, 128), lambda i: (0, 0)),
          pl.BlockSpec((2, blk, 128), lambda i: (0, i, 0)),
          pl.BlockSpec((blk, 128), lambda i: (i, 0)),
      ],
      out_specs=pl.BlockSpec((blk, 128), lambda i: (i, 0)),
      out_shape=jax.ShapeDtypeStruct((n, 128), jnp.bfloat16),
  )(h1, ws2, b2, agg2, inv)


def kernel(features, edge_index, W_self1, W_neigh1, b1, W_self2, W_neigh2,
           b2):
  n, _ = features.shape
  n_edges = edge_index.shape[1]
  src = edge_index[0].astype(jnp.int32).reshape(n_edges // CH, CH)
  dst = edge_index[1].astype(jnp.int32).reshape(n_edges // CH, CH)

  segsum_ext = _make_segsum(n, n_edges, 144, nb=3)
  segsum = _make_segsum(n, n_edges, 128, nb=3)
  score_k = _make_score(n, n_edges, 128)

  y1e = _tc1(features, W_neigh1)
  agg1 = segsum_ext(y1e, src, dst).reshape(2, n, 144)
  h1, y2, inv = _tc2(features, W_self1, b1.reshape(1, 128), W_neigh2, agg1)
  agg2 = segsum(y2, src, dst).reshape(2, n, 128)
  h2 = _tc3(h1, W_self2, b2.reshape(1, 128), agg2, inv)
  return score_k(h2, src, dst).reshape(n_edges)


# speedup vs baseline: 1.4782x; 1.0521x over previous
import functools

import jax
import jax.numpy as jnp
from jax import lax
from jax.experimental import pallas as pl
from jax.experimental.pallas import tpu as pltpu
from jax.experimental.pallas import tpu_sc as plsc

NC = 2
NS = 16
NW = NC * NS

CH = 80
NB = 4


def _sc_mesh():
  return plsc.VectorSubcoreMesh(
      core_axis_name="c", subcore_axis_name="s", num_cores=NC,
      num_subcores=NS)


_SC_PARAMS = pltpu.CompilerParams(
    use_tc_tiling_on_sc=False, needs_layout_passes=False)

def _shuffle_add(v, iot):
  dnums = lax.GatherDimensionNumbers(
      offset_dims=(), collapsed_slice_dims=(0,), start_index_map=(0,))
  for sh in (8, 4, 2, 1):
    idx = jnp.bitwise_xor(iot, jnp.int32(sh)).reshape(16, 1)
    v = v + lax.gather(v, idx, dnums, slice_sizes=(1,),
                       mode=lax.GatherScatterMode.PROMISE_IN_BOUNDS)
  return v


def _zero_2d(ref, rows, width):
  zv = jnp.zeros((16,), jnp.float32)

  @pl.loop(0, rows)
  def _(r):
    for k in range(width // 16):
      ref[r, pl.ds(k * 16, 16)] = zv


def _make_segsum(n_nodes, n_edges, width, nb=2):
  ept = n_edges // NW
  nchunk = ept // CH
  n_outer = (nchunk + nb - 1) // nb
  rows_pt = n_nodes // NS
  nfull, tail = divmod(rows_pt, CH)

  scratch = (
      [pltpu.VMEM((nb, CH), jnp.int32)] * 2
      + [pltpu.VMEM((CH, width), jnp.float32) for _ in range(nb)]
      + [pltpu.VMEM_SHARED((n_nodes, width), jnp.float32)]
      + [pltpu.SemaphoreType.DMA] * (3 * nb)
  )

  @functools.partial(
      pl.kernel,
      out_type=jax.ShapeDtypeStruct((NC * n_nodes, width), jnp.float32),
      mesh=_sc_mesh(),
      scratch_types=scratch,
      compiler_params=_SC_PARAMS,
  )
  def segsum(y_hbm, src2d, dst2d, out_hbm, sring, dring, *rest):
    bufs = rest[:nb]
    agg_sh = rest[nb]
    gsem = rest[nb + 1:2 * nb + 1]
    isems = rest[2 * nb + 1:3 * nb + 1]
    isemd = rest[3 * nb + 1:]
    c = lax.axis_index("c")
    s = lax.axis_index("s")
    wid = c * NS + s
    cbase = wid * nchunk

    _zero_2d(bufs[0], CH, width)
    for r in range(nfull):
      pltpu.sync_copy(bufs[0], agg_sh.at[pl.ds(s * rows_pt + r * CH, CH)])
    if tail:
      pltpu.sync_copy(bufs[0].at[pl.ds(0, tail)],
                      agg_sh.at[pl.ds(s * rows_pt + nfull * CH, tail)])
    plsc.subcore_barrier()

    for b in range(nb):
      pltpu.sync_copy(src2d.at[cbase + b], sring.at[b])
      pltpu.sync_copy(dst2d.at[cbase + b], dring.at[b])
      pltpu.async_copy(y_hbm.at[sring.at[b]], bufs[b], gsem[b])

    @pl.loop(0, n_outer)
    def _(o):
      for b in range(nb):
        i = o * nb + b
        b1 = (b + 1) % nb

        @pl.when(i < nchunk)
        def _(i=i, b=b, b1=b1):
          pltpu.make_async_copy(y_hbm.at[sring.at[b]], bufs[b],
                                gsem[b]).wait()
          j = i + 1

          @pl.when(jnp.logical_and(j >= nb, j < nchunk))
          def _():
            pltpu.make_async_copy(src2d.at[cbase + j], sring.at[b1],
                                  isems[b1]).wait()
            pltpu.make_async_copy(dst2d.at[cbase + j], dring.at[b1],
                                  isemd[b1]).wait()
            pltpu.async_copy(y_hbm.at[sring.at[b1]], bufs[b1], gsem[b1])

          pltpu.sync_copy(bufs[b], agg_sh.at[dring.at[b]], add=True)

          @pl.when(i + nb < nchunk)
          def _():
            pltpu.async_copy(src2d.at[cbase + i + nb], sring.at[b], isems[b])
            pltpu.async_copy(dst2d.at[cbase + i + nb], dring.at[b], isemd[b])

    plsc.subcore_barrier()

    for r in range(nfull):
      row0 = s * rows_pt + r * CH
      pltpu.sync_copy(agg_sh.at[pl.ds(row0, CH)], bufs[0])
      pltpu.sync_copy(bufs[0], out_hbm.at[pl.ds(c * n_nodes + row0, CH)])
    if tail:
      row0 = s * rows_pt + nfull * CH
      pltpu.sync_copy(agg_sh.at[pl.ds(row0, tail)],
                      bufs[0].at[pl.ds(0, tail)])
      pltpu.sync_copy(bufs[0].at[pl.ds(0, tail)],
                      out_hbm.at[pl.ds(c * n_nodes + row0, tail)])

  return segsum


def _make_score(n_nodes, n_edges, width, nb=3):
  ept = n_edges // NW
  nchunk = ept // CH
  n_outer = (nchunk + nb - 1) // nb

  scratch = (
      [pltpu.VMEM((nb, CH), jnp.int32)] * 2
      + [pltpu.VMEM((CH, width), jnp.bfloat16) for _ in range(2 * nb)]
      + [pltpu.VMEM((nchunk, CH), jnp.float32),
         pltpu.VMEM((CH // 16, 16, 17), jnp.float32)]
      + [pltpu.SemaphoreType.DMA] * (4 * nb)
  )

  @functools.partial(
      pl.kernel,
      out_type=jax.ShapeDtypeStruct((n_edges // CH, CH), jnp.float32),
      mesh=_sc_mesh(),
      scratch_types=scratch,
      compiler_params=_SC_PARAMS,
  )
  def score(h_hbm, src2d, dst2d, out_hbm, sring, dring, *rest):
    sbufs = rest[:nb]
    dbufs = rest[nb:2 * nb]
    out_all = rest[2 * nb]
    tmp = rest[2 * nb + 1]
    gsems = rest[2 * nb + 2:3 * nb + 2]
    gsemd = rest[3 * nb + 2:4 * nb + 2]
    isems = rest[4 * nb + 2:5 * nb + 2]
    isemd = rest[5 * nb + 2:]
    c = lax.axis_index("c")
    s = lax.axis_index("s")
    wid = c * NS + s
    cbase = wid * nchunk
    iot = lax.iota(jnp.int32, 16)

    for b in range(nb):
      pltpu.sync_copy(src2d.at[cbase + b], sring.at[b])
      pltpu.sync_copy(dst2d.at[cbase + b], dring.at[b])
      pltpu.async_copy(h_hbm.at[sring.at[b]], sbufs[b], gsems[b])
      pltpu.async_copy(h_hbm.at[dring.at[b]], dbufs[b], gsemd[b])

    @pl.loop(0, n_outer)
    def _(o):
      for b in range(nb):
        i = o * nb + b
        b1 = (b + 1) % nb

        @pl.when(i < nchunk)
        def _(i=i, b=b, b1=b1):
          pltpu.make_async_copy(h_hbm.at[sring.at[b]], sbufs[b],
                                gsems[b]).wait()
          pltpu.make_async_copy(h_hbm.at[dring.at[b]], dbufs[b],
                                gsemd[b]).wait()
          j = i + 1

          @pl.when(jnp.logical_and(j >= nb, j < nchunk))
          def _():
            pltpu.make_async_copy(src2d.at[cbase + j], sring.at[b1],
                                  isems[b1]).wait()
            pltpu.make_async_copy(dst2d.at[cbase + j], dring.at[b1],
                                  isemd[b1]).wait()
            pltpu.async_copy(h_hbm.at[sring.at[b1]], sbufs[b1], gsems[b1])
            pltpu.async_copy(h_hbm.at[dring.at[b1]], dbufs[b1], gsemd[b1])

          @plsc.parallel_loop(0, CH // 16)
          def _(g):
            for el in range(16):
              e = g * 16 + el
              acc = None
              for k in range(width // 32):
                sv = sbufs[b][e, pl.ds(k * 32, 32)]
                dv = dbufs[b][e, pl.ds(k * 32, 32)]
                s0, s1 = plsc.unpack(sv, format=plsc.PackFormat.INTERLEAVED)
                d0, d1 = plsc.unpack(dv, format=plsc.PackFormat.INTERLEAVED)
                p = s0 * d0 + s1 * d1
                acc = p if acc is None else acc + p
              tmp[g, el, pl.ds(0, 16)] = acc
            gv = jnp.full((16,), g, dtype=jnp.int32)
            col = None
            for l in range(16):
              lv = jnp.full((16,), l, dtype=jnp.int32)
              cv = plsc.load_gather(tmp, [gv, iot, lv])
              col = cv if col is None else col + cv
            out_all[i, pl.ds(g * 16, 16)] = col

          @pl.when(i + nb < nchunk)
          def _():
            pltpu.async_copy(src2d.at[cbase + i + nb], sring.at[b], isems[b])
            pltpu.async_copy(dst2d.at[cbase + i + nb], dring.at[b], isemd[b])

    pltpu.sync_copy(out_all, out_hbm.at[pl.ds(cbase, nchunk)])

  return score


def _tc1(x, wn1):
  n, din = x.shape
  blk = 1000

  def body(x_ref, w_ref, o_ref):
    y = jnp.dot(x_ref[...], w_ref[...], preferred_element_type=jnp.float32)
    pad_iota = lax.broadcasted_iota(jnp.int32, (blk, 16), 1)
    extra = jnp.where(pad_iota == 0, 1.0, 0.0).astype(jnp.float32)
    o_ref[...] = jnp.concatenate([y, extra], axis=1)

  return pl.pallas_call(
      body,
      grid=(n // blk,),
      in_specs=[
          pl.BlockSpec((blk, din), lambda i: (i, 0)),
          pl.BlockSpec((din, 128), lambda i: (0, 0)),
      ],
      out_specs=pl.BlockSpec((blk, 144), lambda i: (i, 0)),
      out_shape=jax.ShapeDtypeStruct((n, 144), jnp.float32),
  )(x, wn1)


def _tc2(x, ws1, b1, wn2, ws2, b2, agg1):
  n, din = x.shape
  blk = 1000

  def body(x_ref, ws_ref, b_ref, wn2_ref, ws2_ref, b2_ref, agg_ref,
           ha_ref, y2_ref, inv_ref):
    agg = agg_ref[0] + agg_ref[1]
    deg = agg[:, 128:129]
    inv = 1.0 / jnp.maximum(deg, 1.0)
    mean = agg[:, :128] * inv
    h = jnp.dot(x_ref[...], ws_ref[...], preferred_element_type=jnp.float32)
    h = jnp.maximum(h + mean + b_ref[...], 0.0)
    y2_ref[...] = jnp.dot(h, wn2_ref[...], preferred_element_type=jnp.float32)
    ha_ref[...] = (
        jnp.dot(h, ws2_ref[...], preferred_element_type=jnp.float32)
        + b2_ref[...])
    inv_ref[...] = jnp.broadcast_to(inv, (blk, 128))

  return pl.pallas_call(
      body,
      grid=(n // blk,),
      in_specs=[
          pl.BlockSpec((blk, din), lambda i: (i, 0)),
          pl.BlockSpec((din, 128), lambda i: (0, 0)),
          pl.BlockSpec((1, 128), lambda i: (0, 0)),
          pl.BlockSpec((128, 128), lambda i: (0, 0)),
          pl.BlockSpec((128, 128), lambda i: (0, 0)),
          pl.BlockSpec((1, 128), lambda i: (0, 0)),
          pl.BlockSpec((2, blk, 144), lambda i: (0, i, 0)),
      ],
      out_specs=[
          pl.BlockSpec((blk, 128), lambda i: (i, 0)),
          pl.BlockSpec((blk, 128), lambda i: (i, 0)),
          pl.BlockSpec((blk, 128), lambda i: (i, 0)),
      ],
      out_shape=[
          jax.ShapeDtypeStruct((n, 128), jnp.float32),
          jax.ShapeDtypeStruct((n, 128), jnp.float32),
          jax.ShapeDtypeStruct((n, 128), jnp.float32),
      ],
  )(x, ws1, b1, wn2, ws2, b2, agg1)


def _tc3(ha, agg2, inv):
  n, _ = ha.shape
  blk = 1000

  def body(ha_ref, agg_ref, inv_ref, o_ref):
    agg = agg_ref[0] + agg_ref[1]
    o_ref[...] = (ha_ref[...] + agg * inv_ref[...]).astype(jnp.bfloat16)

  return pl.pallas_call(
      body,
      grid=(n // blk,),
      in_specs=[
          pl.BlockSpec((blk, 128), lambda i: (i, 0)),
          pl.BlockSpec((2, blk, 128), lambda i: (0, i, 0)),
          pl.BlockSpec((blk, 128), lambda i: (i, 0)),
      ],
      out_specs=pl.BlockSpec((blk, 128), lambda i: (i, 0)),
      out_shape=jax.ShapeDtypeStruct((n, 128), jnp.bfloat16),
  )(ha, agg2, inv)


def kernel(features, edge_index, W_self1, W_neigh1, b1, W_self2, W_neigh2,
           b2):
  n, _ = features.shape
  n_edges = edge_index.shape[1]
  src = edge_index[0].astype(jnp.int32).reshape(n_edges // CH, CH)
  dst = edge_index[1].astype(jnp.int32).reshape(n_edges // CH, CH)

  segsum_ext = _make_segsum(n, n_edges, 144, nb=3)
  segsum = _make_segsum(n, n_edges, 128, nb=3)
  score_k = _make_score(n, n_edges, 128)

  y1e = _tc1(features, W_neigh1)
  agg1 = segsum_ext(y1e, src, dst).reshape(2, n, 144)
  ha, y2, inv = _tc2(features, W_self1, b1.reshape(1, 128), W_neigh2,
                     W_self2, b2.reshape(1, 128), agg1)
  agg2 = segsum(y2, src, dst).reshape(2, n, 128)
  h2 = _tc3(ha, agg2, inv)
  return score_k(h2, src, dst).reshape(n_edges)


# speedup vs baseline: 1.4848x; 1.0045x over previous
import functools

import jax
import jax.numpy as jnp
from jax import lax
from jax.experimental import pallas as pl
from jax.experimental.pallas import tpu as pltpu
from jax.experimental.pallas import tpu_sc as plsc

NC = 2
NS = 16
NW = NC * NS

CH = 80
NB = 4


def _sc_mesh():
  return plsc.VectorSubcoreMesh(
      core_axis_name="c", subcore_axis_name="s", num_cores=NC,
      num_subcores=NS)


_SC_PARAMS = pltpu.CompilerParams(
    use_tc_tiling_on_sc=False, needs_layout_passes=False)

def _shuffle_add(v, iot):
  dnums = lax.GatherDimensionNumbers(
      offset_dims=(), collapsed_slice_dims=(0,), start_index_map=(0,))
  for sh in (8, 4, 2, 1):
    idx = jnp.bitwise_xor(iot, jnp.int32(sh)).reshape(16, 1)
    v = v + lax.gather(v, idx, dnums, slice_sizes=(1,),
                       mode=lax.GatherScatterMode.PROMISE_IN_BOUNDS)
  return v


def _zero_2d(ref, rows, width):
  zv = jnp.zeros((16,), jnp.float32)

  @pl.loop(0, rows)
  def _(r):
    for k in range(width // 16):
      ref[r, pl.ds(k * 16, 16)] = zv


def _make_segsum(n_nodes, n_edges, width, nb=2):
  ept = n_edges // NW
  nchunk = ept // CH
  n_outer = (nchunk + nb - 1) // nb
  rows_pt = n_nodes // NS
  nfull, tail = divmod(rows_pt, CH)

  scratch = (
      [pltpu.VMEM((nb, CH), jnp.int32)] * 2
      + [pltpu.VMEM((CH, width), jnp.float32) for _ in range(nb)]
      + [pltpu.VMEM_SHARED((n_nodes, width), jnp.float32)]
      + [pltpu.SemaphoreType.DMA] * (3 * nb)
  )

  @functools.partial(
      pl.kernel,
      out_type=jax.ShapeDtypeStruct((NC * n_nodes, width), jnp.float32),
      mesh=_sc_mesh(),
      scratch_types=scratch,
      compiler_params=_SC_PARAMS,
  )
  def segsum(y_hbm, src2d, dst2d, out_hbm, sring, dring, *rest):
    bufs = rest[:nb]
    agg_sh = rest[nb]
    gsem = rest[nb + 1:2 * nb + 1]
    isems = rest[2 * nb + 1:3 * nb + 1]
    isemd = rest[3 * nb + 1:]
    c = lax.axis_index("c")
    s = lax.axis_index("s")
    wid = c * NS + s
    cbase = wid * nchunk

    _zero_2d(bufs[0], CH, width)
    for r in range(nfull):
      pltpu.sync_copy(bufs[0], agg_sh.at[pl.ds(s * rows_pt + r * CH, CH)])
    if tail:
      pltpu.sync_copy(bufs[0].at[pl.ds(0, tail)],
                      agg_sh.at[pl.ds(s * rows_pt + nfull * CH, tail)])
    plsc.subcore_barrier()

    for b in range(nb):
      pltpu.sync_copy(src2d.at[cbase + b], sring.at[b])
      pltpu.sync_copy(dst2d.at[cbase + b], dring.at[b])
      pltpu.async_copy(y_hbm.at[sring.at[b]], bufs[b], gsem[b])

    @pl.loop(0, n_outer)
    def _(o):
      for b in range(nb):
        i = o * nb + b
        b1 = (b + 1) % nb

        @pl.when(i < nchunk)
        def _(i=i, b=b, b1=b1):
          pltpu.make_async_copy(y_hbm.at[sring.at[b]], bufs[b],
                                gsem[b]).wait()
          j = i + 1

          @pl.when(jnp.logical_and(j >= nb, j < nchunk))
          def _():
            pltpu.make_async_copy(src2d.at[cbase + j], sring.at[b1],
                                  isems[b1]).wait()
            pltpu.make_async_copy(dst2d.at[cbase + j], dring.at[b1],
                                  isemd[b1]).wait()
            pltpu.async_copy(y_hbm.at[sring.at[b1]], bufs[b1], gsem[b1])

          pltpu.sync_copy(bufs[b], agg_sh.at[dring.at[b]], add=True)

          @pl.when(i + nb < nchunk)
          def _():
            pltpu.async_copy(src2d.at[cbase + i + nb], sring.at[b], isems[b])
            pltpu.async_copy(dst2d.at[cbase + i + nb], dring.at[b], isemd[b])

    plsc.subcore_barrier()

    for r in range(nfull):
      row0 = s * rows_pt + r * CH
      pltpu.sync_copy(agg_sh.at[pl.ds(row0, CH)], bufs[0])
      pltpu.sync_copy(bufs[0], out_hbm.at[pl.ds(c * n_nodes + row0, CH)])
    if tail:
      row0 = s * rows_pt + nfull * CH
      pltpu.sync_copy(agg_sh.at[pl.ds(row0, tail)],
                      bufs[0].at[pl.ds(0, tail)])
      pltpu.sync_copy(bufs[0].at[pl.ds(0, tail)],
                      out_hbm.at[pl.ds(c * n_nodes + row0, tail)])

  return segsum


def _make_score(n_nodes, n_edges, width, nb=3):
  ept = n_edges // NW
  nchunk = ept // CH
  n_outer = (nchunk + nb - 1) // nb

  scratch = (
      [pltpu.VMEM((nb, CH), jnp.int32)] * 2
      + [pltpu.VMEM((CH, width), jnp.bfloat16) for _ in range(2 * nb)]
      + [pltpu.VMEM((nchunk, CH), jnp.float32),
         pltpu.VMEM((CH // 16, 16, 17), jnp.float32)]
      + [pltpu.SemaphoreType.DMA] * (4 * nb)
  )

  @functools.partial(
      pl.kernel,
      out_type=jax.ShapeDtypeStruct((n_edges // CH, CH), jnp.float32),
      mesh=_sc_mesh(),
      scratch_types=scratch,
      compiler_params=_SC_PARAMS,
  )
  def score(h_hbm, src2d, dst2d, out_hbm, sring, dring, *rest):
    sbufs = rest[:nb]
    dbufs = rest[nb:2 * nb]
    out_all = rest[2 * nb]
    tmp = rest[2 * nb + 1]
    gsems = rest[2 * nb + 2:3 * nb + 2]
    gsemd = rest[3 * nb + 2:4 * nb + 2]
    isems = rest[4 * nb + 2:5 * nb + 2]
    isemd = rest[5 * nb + 2:]
    c = lax.axis_index("c")
    s = lax.axis_index("s")
    wid = c * NS + s
    cbase = wid * nchunk
    iot = lax.iota(jnp.int32, 16)

    for b in range(nb):
      pltpu.sync_copy(src2d.at[cbase + b], sring.at[b])
      pltpu.sync_copy(dst2d.at[cbase + b], dring.at[b])
      pltpu.async_copy(h_hbm.at[sring.at[b]], sbufs[b], gsems[b])
      pltpu.async_copy(h_hbm.at[dring.at[b]], dbufs[b], gsemd[b])

    @pl.loop(0, n_outer)
    def _(o):
      for b in range(nb):
        i = o * nb + b
        b1 = (b + 1) % nb

        @pl.when(i < nchunk)
        def _(i=i, b=b, b1=b1):
          pltpu.make_async_copy(h_hbm.at[sring.at[b]], sbufs[b],
                                gsems[b]).wait()
          pltpu.make_async_copy(h_hbm.at[dring.at[b]], dbufs[b],
                                gsemd[b]).wait()
          j = i + 1

          @pl.when(jnp.logical_and(j >= nb, j < nchunk))
          def _():
            pltpu.make_async_copy(src2d.at[cbase + j], sring.at[b1],
                                  isems[b1]).wait()
            pltpu.make_async_copy(dst2d.at[cbase + j], dring.at[b1],
                                  isemd[b1]).wait()
            pltpu.async_copy(h_hbm.at[sring.at[b1]], sbufs[b1], gsems[b1])
            pltpu.async_copy(h_hbm.at[dring.at[b1]], dbufs[b1], gsemd[b1])

          @plsc.parallel_loop(0, CH // 16)
          def _(g):
            for el in range(16):
              e = g * 16 + el
              acc = None
              for k in range(width // 32):
                sv = sbufs[b][e, pl.ds(k * 32, 32)]
                dv = dbufs[b][e, pl.ds(k * 32, 32)]
                s0, s1 = plsc.unpack(sv, format=plsc.PackFormat.INTERLEAVED)
                d0, d1 = plsc.unpack(dv, format=plsc.PackFormat.INTERLEAVED)
                p = s0 * d0 + s1 * d1
                acc = p if acc is None else acc + p
              tmp[g, el, pl.ds(0, 16)] = acc
            gv = jnp.full((16,), g, dtype=jnp.int32)
            col = None
            for l in range(16):
              lv = jnp.full((16,), l, dtype=jnp.int32)
              cv = plsc.load_gather(tmp, [gv, iot, lv])
              col = cv if col is None else col + cv
            out_all[i, pl.ds(g * 16, 16)] = col

          @pl.when(i + nb < nchunk)
          def _():
            pltpu.async_copy(src2d.at[cbase + i + nb], sring.at[b], isems[b])
            pltpu.async_copy(dst2d.at[cbase + i + nb], dring.at[b], isemd[b])

    pltpu.sync_copy(out_all, out_hbm.at[pl.ds(cbase, nchunk)])

  return score


def _tc1(x, wn1):
  n, din = x.shape
  blk = 1000

  def body(x_ref, w_ref, o_ref):
    y = jnp.dot(x_ref[...], w_ref[...], preferred_element_type=jnp.float32)
    pad_iota = lax.broadcasted_iota(jnp.int32, (blk, 16), 1)
    extra = jnp.where(pad_iota == 0, 1.0, 0.0).astype(jnp.float32)
    o_ref[...] = jnp.concatenate([y, extra], axis=1)

  return pl.pallas_call(
      body,
      grid=(n // blk,),
      in_specs=[
          pl.BlockSpec((blk, din), lambda i: (i, 0)),
          pl.BlockSpec((din, 128), lambda i: (0, 0)),
      ],
      out_specs=pl.BlockSpec((blk, 144), lambda i: (i, 0)),
      out_shape=jax.ShapeDtypeStruct((n, 144), jnp.float32),
  )(x, wn1)


def _tc2(x, ws1, b1, wn2, agg1):
  n, din = x.shape
  blk = 1000

  def body(x_ref, ws_ref, b_ref, wn2_ref, agg_ref, h_ref, y2_ref, inv_ref):
    agg = agg_ref[0] + agg_ref[1]
    deg = agg[:, 128:129]
    inv = 1.0 / jnp.maximum(deg, 1.0)
    mean = agg[:, :128] * inv
    h = jnp.dot(x_ref[...], ws_ref[...], preferred_element_type=jnp.float32)
    h = jnp.maximum(h + mean + b_ref[...], 0.0)
    h_ref[...] = h
    y2_ref[...] = jnp.dot(h, wn2_ref[...], preferred_element_type=jnp.float32)
    inv_ref[...] = jnp.broadcast_to(inv, (blk, 128))

  return pl.pallas_call(
      body,
      grid=(n // blk,),
      in_specs=[
          pl.BlockSpec((blk, din), lambda i: (i, 0)),
          pl.BlockSpec((din, 128), lambda i: (0, 0)),
          pl.BlockSpec((1, 128), lambda i: (0, 0)),
          pl.BlockSpec((128, 128), lambda i: (0, 0)),
          pl.BlockSpec((2, blk, 144), lambda i: (0, i, 0)),
      ],
      out_specs=[
          pl.BlockSpec((blk, 128), lambda i: (i, 0)),
          pl.BlockSpec((blk, 128), lambda i: (i, 0)),
          pl.BlockSpec((blk, 128), lambda i: (i, 0)),
      ],
      out_shape=[
          jax.ShapeDtypeStruct((n, 128), jnp.float32),
          jax.ShapeDtypeStruct((n, 128), jnp.float32),
          jax.ShapeDtypeStruct((n, 128), jnp.float32),
      ],
  )(x, ws1, b1, wn2, agg1)


def _tc3(h1, ws2, b2, agg2, inv):
  n, din = h1.shape
  blk = 1000

  def body(h1_ref, ws_ref, b_ref, agg_ref, inv_ref, o_ref):
    agg = agg_ref[0] + agg_ref[1]
    h = jnp.dot(h1_ref[...], ws_ref[...], preferred_element_type=jnp.float32)
    o_ref[...] = (h + agg * inv_ref[...] + b_ref[...]).astype(jnp.bfloat16)

  return pl.pallas_call(
      body,
      grid=(n // blk,),
      in_specs=[
          pl.BlockSpec((blk, din), lambda i: (i, 0)),
          pl.BlockSpec((din, 128), lambda i: (0, 0)),
          pl.BlockSpec((1, 128), lambda i: (0, 0)),
          pl.BlockSpec((2, blk, 128), lambda i: (0, i, 0)),
          pl.BlockSpec((blk, 128), lambda i: (i, 0)),
      ],
      out_specs=pl.BlockSpec((blk, 128), lambda i: (i, 0)),
      out_shape=jax.ShapeDtypeStruct((n, 128), jnp.bfloat16),
  )(h1, ws2, b2, agg2, inv)


def kernel(features, edge_index, W_self1, W_neigh1, b1, W_self2, W_neigh2,
           b2):
  n, _ = features.shape
  n_edges = edge_index.shape[1]
  src = edge_index[0].astype(jnp.int32).reshape(n_edges // CH, CH)
  dst = edge_index[1].astype(jnp.int32).reshape(n_edges // CH, CH)

  segsum_ext = _make_segsum(n, n_edges, 144, nb=3)
  segsum = _make_segsum(n, n_edges, 128, nb=3)
  score_k = _make_score(n, n_edges, 128)

  y1e = _tc1(features, W_neigh1)
  agg1 = segsum_ext(y1e, src, dst).reshape(2, n, 144)
  h1, y2, inv = _tc2(features, W_self1, b1.reshape(1, 128), W_neigh2, agg1)
  agg2 = segsum(y2, src, dst).reshape(2, n, 128)
  h2 = _tc3(h1, W_self2, b2.reshape(1, 128), agg2, inv)
  return score_k(h2, src, dst).reshape(n_edges)
